# Initial kernel scaffold; baseline (speedup 1.0000x reference)
import jax
import jax.numpy as jnp
from jax.experimental import pallas as pl


def kernel(inputs, hidden, edge_logits, send_edges, recv_edges, msg_fc1_w, msg_fc1_b, msg_fc2_w, msg_fc2_b, hidden_r_w, hidden_i_w, hidden_h_w, input_r_w, input_r_b, input_i_w, input_i_b, input_n_w, input_n_b, out_w1, out_b1, out_w2, out_b2, proj_loc_w, proj_loc_b, proj_scale_w, proj_scale_b):
    raise NotImplementedError("write your pallas kernel here")


# speedup vs baseline: 4.8431x; 4.8431x over previous
import functools

import jax
import jax.numpy as jnp
from jax import lax
from jax.experimental import pallas as pl
from jax.experimental.pallas import tpu as pltpu
from jax.experimental.pallas import tpu_sc as plsc

B = 4
N = 1000
E = 64000
H = 64
IN = 8
L = 2
ET = 2
TAU = 0.5

BE = B * E

_INTERPRET = False


def _edges_body(l0, l1, g0, g1, e0, e1):
    a = (l0[...] + g0[...]) / TAU
    b = (l1[...] + g1[...]) / TAU
    m = jnp.maximum(a, b)
    ea = jnp.exp(a - m)
    eb = jnp.exp(b - m)
    s = ea + eb
    y0 = ea / s
    y1 = eb / s
    h1 = (b > a).astype(jnp.float32)
    h0 = 1.0 - h1
    e0[...] = (h0 - y0) + y0
    e1[...] = (h1 - y1) + y1


def _edges_tc(l0, l1, g0, g1):
    return pl.pallas_call(
        _edges_body,
        out_shape=(
            jax.ShapeDtypeStruct((2000, 128), jnp.float32),
            jax.ShapeDtypeStruct((2000, 128), jnp.float32),
        ),
        interpret=_INTERPRET,
    )(l0, l1, g0, g1)


def _node0_body(h2, w1a, w1b, b1, x2, wr, br, wi, bi, wn, bn,
                hr, hs, ir, ii, inn):
    h = h2[...]
    hr[...] = jnp.dot(h, w1a[...], preferred_element_type=jnp.float32) + b1[...]
    hs[...] = jnp.dot(h, w1b[...], preferred_element_type=jnp.float32)
    x = x2[...]
    ir[...] = jnp.dot(x, wr[...], preferred_element_type=jnp.float32) + br[...]
    ii[...] = jnp.dot(x, wi[...], preferred_element_type=jnp.float32) + bi[...]
    inn[...] = jnp.dot(x, wn[...], preferred_element_type=jnp.float32) + bn[...]


def _node0_tc(h2, w1a, w1b, b1, x2, wr, br, wi, bi, wn, bn):
    f = jax.ShapeDtypeStruct
    return pl.pallas_call(
        _node0_body,
        out_shape=(
            f((B * N, H), jnp.float32), f((B * N, H), jnp.float32),
            f((B * N, H), jnp.float32), f((B * N, H), jnp.float32),
            f((B * N, H), jnp.float32),
        ),
        interpret=_INTERPRET,
    )(h2, w1a, w1b, b1, x2, wr, br, wi, bi, wn, bn)


_FC2_BLK = 2048


def _fc2_body(zr, zs, w2, b2, msk, out):
    m = jnp.tanh(zr[...] + zs[...])
    t = jnp.dot(m, w2[...], preferred_element_type=jnp.float32) + b2[...]
    out[...] = jnp.tanh(t) * msk[...]


def _fc2_tc(zr, zs, w2, b2, mask2d):
    grid = BE // _FC2_BLK
    return pl.pallas_call(
        _fc2_body,
        grid=(grid,),
        in_specs=[
            pl.BlockSpec((_FC2_BLK, H), lambda i: (i, 0)),
            pl.BlockSpec((_FC2_BLK, H), lambda i: (i, 0)),
            pl.BlockSpec((H, H), lambda i: (0, 0)),
            pl.BlockSpec((1, H), lambda i: (0, 0)),
            pl.BlockSpec((_FC2_BLK, 1), lambda i: (i, 0)),
        ],
        out_specs=pl.BlockSpec((_FC2_BLK, H), lambda i: (i, 0)),
        out_shape=jax.ShapeDtypeStruct((BE, H), jnp.float32),
        interpret=_INTERPRET,
    )(zr, zs, w2, b2, mask2d)


def _comb_body(p0, p1, dp, w1a, w1b, b1, agg0, hr, hs, dout):
    d = dp[0] + dp[1]
    dcol = d[:, 0:1]
    dcol = jnp.where(dcol == 0.0, 1.0, dcol)
    d4 = jnp.concatenate([dcol, dcol, dcol, dcol], axis=0)
    a0 = (p0[...] + p1[...]) / d4
    agg0[...] = a0
    hr[...] = jnp.dot(a0, w1a[...], preferred_element_type=jnp.float32) + b1[...]
    hs[...] = jnp.dot(a0, w1b[...], preferred_element_type=jnp.float32)
    dout[...] = dcol


def _comb_tc(p0, p1, dp, w1a, w1b, b1):
    f = jax.ShapeDtypeStruct
    return pl.pallas_call(
        _comb_body,
        out_shape=(
            f((B * N, H), jnp.float32), f((B * N, H), jnp.float32),
            f((B * N, H), jnp.float32), f((N, 1), jnp.float32),
        ),
        interpret=_INTERPRET,
    )(p0, p1, dp, w1a, w1b, b1)


def _gru_body(pa, pb, dcol, agg0, ir, ii, inn, h2, wr, wi, wh,
              ow1, ob1, ow2, ob2, hn_out, pred_out):
    dc = dcol[...]
    d4 = jnp.concatenate([dc, dc, dc, dc], axis=0)
    agg1 = (pa[...] + pb[...]) / d4
    am = jnp.concatenate([agg0[...], agg1], axis=1)
    r = jax.nn.sigmoid(ir[...] + jnp.dot(am, wr[...], preferred_element_type=jnp.float32))
    ig = jax.nn.sigmoid(ii[...] + jnp.dot(am, wi[...], preferred_element_type=jnp.float32))
    ng = jnp.tanh(inn[...] + r * jnp.dot(am, wh[...], preferred_element_type=jnp.float32))
    hn = (1.0 - ig) * ng + ig * h2[...]
    hn_out[...] = hn
    h1 = jax.nn.relu(jnp.dot(hn, ow1[...], preferred_element_type=jnp.float32) + ob1[...])
    pred_out[...] = jax.nn.relu(jnp.dot(h1, ow2[...], preferred_element_type=jnp.float32) + ob2[...])


def _gru_tc(pa, pb, dcol, agg0, ir, ii, inn, h2, wr, wi, wh, ow1, ob1, ow2, ob2):
    f = jax.ShapeDtypeStruct
    return pl.pallas_call(
        _gru_body,
        out_shape=(f((B * N, H), jnp.float32), f((B * N, H), jnp.float32)),
        interpret=_INTERPRET,
    )(pa, pb, dcol, agg0, ir, ii, inn, h2, wr, wi, wh, ow1, ob1, ow2, ob2)


_PROJ_KC = 1280


def _proj_body(flat, wl, ws, bl, bs, loc, scl):
    i = pl.program_id(0)
    nsteps = pl.num_programs(0)
    fb = flat[...]
    pl_part = jnp.dot(fb, wl[...], preferred_element_type=jnp.float32)
    ps_part = jnp.dot(fb, ws[...], preferred_element_type=jnp.float32)

    @pl.when(i == 0)
    def _():
        loc[...] = jnp.zeros_like(loc)
        scl[...] = jnp.zeros_like(scl)

    loc[...] += pl_part
    scl[...] += ps_part

    @pl.when(i == nsteps - 1)
    def _():
        loc[...] += bl[...]
        s = scl[...] + bs[...]
        scl[...] = jnp.log1p(jnp.exp(-jnp.abs(s))) + jax.nn.relu(s)


def _proj_tc(flat, wl, ws, bl, bs):
    grid = (N * H) // _PROJ_KC
    f = jax.ShapeDtypeStruct
    return pl.pallas_call(
        _proj_body,
        grid=(grid,),
        in_specs=[
            pl.BlockSpec((B, _PROJ_KC), lambda i: (0, i)),
            pl.BlockSpec((_PROJ_KC, N), lambda i: (i, 0)),
            pl.BlockSpec((_PROJ_KC, N), lambda i: (i, 0)),
            pl.BlockSpec((1, N), lambda i: (0, 0)),
            pl.BlockSpec((1, N), lambda i: (0, 0)),
        ],
        out_specs=(
            pl.BlockSpec((B, N), lambda i: (0, 0)),
            pl.BlockSpec((B, N), lambda i: (0, 0)),
        ),
        out_shape=(f((B, N), jnp.float32), f((B, N), jnp.float32)),
        interpret=_INTERPRET,
    )(flat, wl, ws, bl, bs)


_NC = 2
_NS = 16
_NW = _NC * _NS
_EPT = E // _NW
_CH = 400
_NCH = _EPT // _CH
_SUB = 80
_NSUB = _CH // _SUB

_sc_mesh = plsc.VectorSubcoreMesh(core_axis_name="c", subcore_axis_name="s")


@functools.partial(
    pl.kernel,
    out_type=(
        jax.ShapeDtypeStruct((BE, H), jnp.float32),
        jax.ShapeDtypeStruct((BE, H), jnp.float32),
        jax.ShapeDtypeStruct((2, 1024, 16), jnp.float32),
    ),
    mesh=_sc_mesh,
    compiler_params=pltpu.CompilerParams(use_tc_tiling_on_sc=False),
    scratch_types=[
        pltpu.VMEM((_CH,), jnp.int32),
        pltpu.VMEM((_CH,), jnp.int32),
        pltpu.VMEM((_NSUB, _SUB), jnp.int32),
        pltpu.VMEM((_NSUB, _SUB), jnp.int32),
        pltpu.VMEM((_CH, H), jnp.float32),
        pltpu.VMEM((_CH, H), jnp.float32),
        pltpu.VMEM((_SUB, 16), jnp.float32),
        pltpu.VMEM_SHARED((1024, 16), jnp.float32),
        pltpu.SemaphoreType.DMA,
    ],
)
def _sc_gather(tabr, tabs, recv, send, ones_h, zer_h, zr, zs, degp,
               rawr, raws, idx2r, idx2s, bufr, bufs, onesv, dacc, sem):
    c = lax.axis_index("c")
    s = lax.axis_index("s")
    w = c * _NS + s
    pltpu.sync_copy(ones_h, onesv)
    pltpu.sync_copy(zer_h, dacc.at[pl.ds(s * 64, 64)])
    plsc.subcore_barrier()
    for b in range(B):
        for ch in range(_NCH):
            e0 = w * _EPT + ch * _CH
            pltpu.sync_copy(recv.at[pl.ds(e0, _CH)], rawr)
            pltpu.sync_copy(send.at[pl.ds(e0, _CH)], raws)
            for j in range(_NSUB):
                for i in range(_SUB // 16):
                    sl = pl.ds(j * _SUB + i * 16, 16)
                    dl = pl.ds(i * 16, 16)
                    idx2r[j, dl] = rawr[sl] + b * N
                    idx2s[j, dl] = raws[sl] + b * N
            cps = []
            for j in range(_NSUB):
                dst = pl.ds(j * _SUB, _SUB)
                cps.append(pltpu.async_copy(tabr.at[idx2r.at[j]], bufr.at[dst], sem))
                cps.append(pltpu.async_copy(tabs.at[idx2s.at[j]], bufs.at[dst], sem))
            for cp in cps:
                cp.wait()
            row0 = b * E + e0
            pltpu.sync_copy(bufr, zr.at[pl.ds(row0, _CH)])
            pltpu.sync_copy(bufs, zs.at[pl.ds(row0, _CH)])
            if b == 0:
                for j in range(_NSUB):
                    pltpu.sync_copy(onesv, dacc.at[idx2r.at[j]], add=True)
    plsc.subcore_barrier()
    pltpu.sync_copy(dacc.at[pl.ds(s * 64, 64)], degp.at[c, pl.ds(s * 64, 64)])


@functools.partial(
    pl.kernel,
    out_type=jax.ShapeDtypeStruct((2, 4096, H), jnp.float32),
    mesh=_sc_mesh,
    compiler_params=pltpu.CompilerParams(use_tc_tiling_on_sc=False),
    scratch_types=[
        pltpu.VMEM((_CH,), jnp.int32),
        pltpu.VMEM((_NSUB, _SUB), jnp.int32),
        pltpu.VMEM((_CH, H), jnp.float32),
        pltpu.VMEM_SHARED((4096, H), jnp.float32),
    ],
)
def _sc_scatter(m2, recv, zer_h, pout, rawr, idx2, bufm, macc):
    c = lax.axis_index("c")
    s = lax.axis_index("s")
    w = c * _NS + s
    pltpu.sync_copy(zer_h, macc.at[pl.ds(s * 256, 256)])
    plsc.subcore_barrier()
    for b in range(B):
        for ch in range(_NCH):
            e0 = w * _EPT + ch * _CH
            pltpu.sync_copy(recv.at[pl.ds(e0, _CH)], rawr)
            pltpu.sync_copy(m2.at[pl.ds(b * E + e0, _CH)], bufm)
            for j in range(_NSUB):
                for i in range(_SUB // 16):
                    idx2[j, pl.ds(i * 16, 16)] = rawr[pl.ds(j * _SUB + i * 16, 16)] + b * 1024
            for j in range(_NSUB):
                pltpu.sync_copy(bufm.at[pl.ds(j * _SUB, _SUB)], macc.at[idx2.at[j]], add=True)
    plsc.subcore_barrier()
    pltpu.sync_copy(macc.at[pl.ds(s * 256, 256)], pout.at[c, pl.ds(s * 256, 256)])


def kernel(inputs, hidden, edge_logits, send_edges, recv_edges,
           msg_fc1_w, msg_fc1_b, msg_fc2_w, msg_fc2_b,
           hidden_r_w, hidden_i_w, hidden_h_w,
           input_r_w, input_r_b, input_i_w, input_i_b, input_n_w, input_n_b,
           out_w1, out_b1, out_w2, out_b2,
           proj_loc_w, proj_loc_b, proj_scale_w, proj_scale_b):
    f32 = jnp.float32

    g = jax.random.gumbel(jax.random.key(42), (BE, ET), dtype=f32)
    l0 = edge_logits[:, :, 0].reshape(2000, 128)
    l1 = edge_logits[:, :, 1].reshape(2000, 128)
    g0 = g[:, 0].reshape(2000, 128)
    g1 = g[:, 1].reshape(2000, 128)
    e0, e1 = _edges_tc(l0, l1, g0, g1)
    edges = jnp.stack([e0.reshape(B, E), e1.reshape(B, E)], axis=-1)
    mask2d = e1.reshape(BE, 1)

    h2 = hidden.reshape(B * N, H)
    x2 = inputs.reshape(B * N, IN)
    w1 = msg_fc1_w[0, 1]
    hr0, hs0, ir, ii, inn = _node0_tc(
        h2, w1[:H], w1[H:], msg_fc1_b[0, 1].reshape(1, H), x2,
        input_r_w, input_r_b.reshape(1, H),
        input_i_w, input_i_b.reshape(1, H),
        input_n_w, input_n_b.reshape(1, H))

    ones16 = jnp.ones((_SUB, 16), f32)
    zer16 = jnp.zeros((64, 16), f32)
    zer64 = jnp.zeros((256, H), f32)

    zr0, zs0, degp4 = _sc_gather(hr0, hs0, recv_edges, send_edges, ones16, zer16)
    m2_0 = _fc2_tc(zr0, zs0, msg_fc2_w[0, 1], msg_fc2_b[0, 1].reshape(1, H), mask2d)
    pout0 = _sc_scatter(m2_0, recv_edges, zer64)
    p0 = pout0.reshape(2, B, 1024, H)[:, :, :N, :].reshape(2, B * N, H)
    degp = degp4[:, :N, :]

    w1_1 = msg_fc1_w[1, 1]
    agg0, hr1, hs1, dcol = _comb_tc(
        p0[0], p0[1], degp, w1_1[:H], w1_1[H:], msg_fc1_b[1, 1].reshape(1, H))

    zr1, zs1, _ = _sc_gather(hr1, hs1, recv_edges, send_edges, ones16, zer16)
    m2_1 = _fc2_tc(zr1, zs1, msg_fc2_w[1, 1], msg_fc2_b[1, 1].reshape(1, H), mask2d)
    pout1 = _sc_scatter(m2_1, recv_edges, zer64)
    p1 = pout1.reshape(2, B, 1024, H)[:, :, :N, :].reshape(2, B * N, H)
    p1a, p1b = p1[0], p1[1]

    hn2, pred2 = _gru_tc(p1a, p1b, dcol, agg0, ir, ii, inn, h2,
                         hidden_r_w, hidden_i_w, hidden_h_w,
                         out_w1, out_b1.reshape(1, H), out_w2, out_b2.reshape(1, H))
    hidden_new = hn2.reshape(B, N, H)

    flat = pred2.reshape(B, N * H)
    loc, scale = _proj_tc(flat, proj_loc_w, proj_scale_w,
                          proj_loc_b.reshape(1, N), proj_scale_b.reshape(1, N))

    return ((loc, scale), hidden_new, edges)


# speedup vs baseline: 10.5566x; 2.1797x over previous
import functools

import jax
import jax.numpy as jnp
from jax import lax
from jax.experimental import pallas as pl
from jax.experimental.pallas import tpu as pltpu
from jax.experimental.pallas import tpu_sc as plsc

B = 4
N = 1000
E = 64000
H = 64
IN = 8
L = 2
ET = 2
TAU = 0.5

BE = B * E
BEH = BE // 2


def _edges_body(l0, l1, g0, g1, e0, e1, mb):
    a = (l0[...] + g0[...]) / TAU
    b = (l1[...] + g1[...]) / TAU
    m = jnp.maximum(a, b)
    ea = jnp.exp(a - m)
    eb = jnp.exp(b - m)
    s = ea + eb
    y0 = ea / s
    y1 = eb / s
    hard1 = b > a
    h1 = hard1.astype(jnp.float32)
    h0 = 1.0 - h1
    e0[...] = (h0 - y0) + y0
    e1[...] = (h1 - y1) + y1
    mb[...] = hard1.astype(jnp.int32)


def _edges_tc(l0, l1, g0, g1):
    return pl.pallas_call(
        _edges_body,
        out_shape=(
            jax.ShapeDtypeStruct((2000, 128), jnp.float32),
            jax.ShapeDtypeStruct((2000, 128), jnp.float32),
            jax.ShapeDtypeStruct((2000, 128), jnp.int32),
        ),
    )(l0, l1, g0, g1)


def _node0_body(h2, w1a, w1b, b1, x2, wr, br, wi, bi, wn, bn,
                hr, hs, ir, ii, inn):
    h = h2[...]
    hr[...] = jnp.dot(h, w1a[...], preferred_element_type=jnp.float32) + b1[...]
    hs[...] = jnp.dot(h, w1b[...], preferred_element_type=jnp.float32)
    x = x2[...]
    ir[...] = jnp.dot(x, wr[...], preferred_element_type=jnp.float32) + br[...]
    ii[...] = jnp.dot(x, wi[...], preferred_element_type=jnp.float32) + bi[...]
    inn[...] = jnp.dot(x, wn[...], preferred_element_type=jnp.float32) + bn[...]


def _node0_tc(h2, w1a, w1b, b1, x2, wr, br, wi, bi, wn, bn):
    f = jax.ShapeDtypeStruct
    return pl.pallas_call(
        _node0_body,
        out_shape=(
            f((B * N, H), jnp.float32), f((B * N, H), jnp.float32),
            f((B * N, H), jnp.float32), f((B * N, H), jnp.float32),
            f((B * N, H), jnp.float32),
        ),
    )(h2, w1a, w1b, b1, x2, wr, br, wi, bi, wn, bn)


_FC2_BLK = 1024


def _fc2_body(zr, zs, w2d, b2d, out):
    m = jnp.tanh(zr[...] + zs[...])
    t = jnp.dot(m, w2d[...], preferred_element_type=jnp.float32) + b2d[...]
    out[...] = jnp.tanh(t)


def _fc2_tc(zr, zs, w2d, b2d):
    grid = BEH // _FC2_BLK
    return pl.pallas_call(
        _fc2_body,
        grid=(grid,),
        in_specs=[
            pl.BlockSpec((_FC2_BLK, 128), lambda i: (i, 0)),
            pl.BlockSpec((_FC2_BLK, 128), lambda i: (i, 0)),
            pl.BlockSpec((128, 128), lambda i: (0, 0)),
            pl.BlockSpec((1, 128), lambda i: (0, 0)),
        ],
        out_specs=pl.BlockSpec((_FC2_BLK, 128), lambda i: (i, 0)),
        out_shape=jax.ShapeDtypeStruct((BEH, 128), jnp.float32),
    )(zr, zs, w2d, b2d)


def _comb_body(p0, p1, dp, w1a, w1b, b1, agg0, hr, hs, dout):
    d = dp[0] + dp[1]
    dcol = d[:, 0:1]
    dcol = jnp.where(dcol == 0.0, 1.0, dcol)
    d4 = jnp.concatenate([dcol, dcol, dcol, dcol], axis=0)
    a0 = (p0[...] + p1[...]) / d4
    agg0[...] = a0
    hr[...] = jnp.dot(a0, w1a[...], preferred_element_type=jnp.float32) + b1[...]
    hs[...] = jnp.dot(a0, w1b[...], preferred_element_type=jnp.float32)
    dout[...] = dcol


def _comb_tc(p0, p1, dp, w1a, w1b, b1):
    f = jax.ShapeDtypeStruct
    return pl.pallas_call(
        _comb_body,
        out_shape=(
            f((B * N, H), jnp.float32), f((B * N, H), jnp.float32),
            f((B * N, H), jnp.float32), f((N, 1), jnp.float32),
        ),
    )(p0, p1, dp, w1a, w1b, b1)


def _gru_body(pa, pb, dcol, agg0, ir, ii, inn, h2, wr, wi, wh,
              ow1, ob1, ow2, ob2, hn_out, pred_out):
    dc = dcol[...]
    d4 = jnp.concatenate([dc, dc, dc, dc], axis=0)
    agg1 = (pa[...] + pb[...]) / d4
    am = jnp.concatenate([agg0[...], agg1], axis=1)
    r = jax.nn.sigmoid(ir[...] + jnp.dot(am, wr[...], preferred_element_type=jnp.float32))
    ig = jax.nn.sigmoid(ii[...] + jnp.dot(am, wi[...], preferred_element_type=jnp.float32))
    ng = jnp.tanh(inn[...] + r * jnp.dot(am, wh[...], preferred_element_type=jnp.float32))
    hn = (1.0 - ig) * ng + ig * h2[...]
    hn_out[...] = hn
    h1 = jax.nn.relu(jnp.dot(hn, ow1[...], preferred_element_type=jnp.float32) + ob1[...])
    pred_out[...] = jax.nn.relu(jnp.dot(h1, ow2[...], preferred_element_type=jnp.float32) + ob2[...])


def _gru_tc(pa, pb, dcol, agg0, ir, ii, inn, h2, wr, wi, wh, ow1, ob1, ow2, ob2):
    f = jax.ShapeDtypeStruct
    return pl.pallas_call(
        _gru_body,
        out_shape=(f((B * N, H), jnp.float32), f((B * N, H), jnp.float32)),
    )(pa, pb, dcol, agg0, ir, ii, inn, h2, wr, wi, wh, ow1, ob1, ow2, ob2)


_PROJ_KC = 1280


def _proj_body(wlt, wst, ft, bl, bs, loc, scl):
    i = pl.program_id(0)
    nsteps = pl.num_programs(0)
    fb = ft[...]
    pl_part = jnp.dot(wlt[...], fb, preferred_element_type=jnp.float32)
    ps_part = jnp.dot(wst[...], fb, preferred_element_type=jnp.float32)

    @pl.when(i == 0)
    def _():
        loc[...] = jnp.zeros_like(loc)
        scl[...] = jnp.zeros_like(scl)

    loc[...] += pl_part
    scl[...] += ps_part

    @pl.when(i == nsteps - 1)
    def _():
        loc[...] += bl[...]
        s = scl[...] + bs[...]
        scl[...] = jnp.log1p(jnp.exp(-jnp.abs(s))) + jax.nn.relu(s)


def _proj_tc(wlt, wst, ft, bl, bs):
    grid = (N * H) // _PROJ_KC
    f = jax.ShapeDtypeStruct
    return pl.pallas_call(
        _proj_body,
        grid=(grid,),
        in_specs=[
            pl.BlockSpec((N, _PROJ_KC), lambda i: (0, i)),
            pl.BlockSpec((N, _PROJ_KC), lambda i: (0, i)),
            pl.BlockSpec((_PROJ_KC, B), lambda i: (i, 0)),
            pl.BlockSpec((N, 1), lambda i: (0, 0)),
            pl.BlockSpec((N, 1), lambda i: (0, 0)),
        ],
        out_specs=(
            pl.BlockSpec((N, B), lambda i: (0, 0)),
            pl.BlockSpec((N, B), lambda i: (0, 0)),
        ),
        out_shape=(f((N, B), jnp.float32), f((N, B), jnp.float32)),
    )(wlt, wst, ft, bl, bs)


_NC = 2
_NS = 16
_NW = _NC * _NS
_EPT = E // _NW
_CH = 400
_NCH = _EPT // _CH
_SUB = 80
_NSUB = _CH // _SUB

_sc_mesh = plsc.VectorSubcoreMesh(core_axis_name="c", subcore_axis_name="s")


def _make_sc_gather(do_deg):
    out_type = [
        jax.ShapeDtypeStruct((BE, H), jnp.float32),
        jax.ShapeDtypeStruct((BE, H), jnp.float32),
    ]
    scratch = [
        pltpu.VMEM((_EPT,), jnp.int32),
        pltpu.VMEM((_EPT,), jnp.int32),
        pltpu.VMEM((2, _NSUB, _SUB), jnp.int32),
        pltpu.VMEM((2, _NSUB, _SUB), jnp.int32),
        pltpu.VMEM((2, _CH, H), jnp.float32),
        pltpu.VMEM((2, _CH, H), jnp.float32),
        pltpu.SemaphoreType.DMA,
        pltpu.SemaphoreType.DMA,
    ]
    if do_deg:
        out_type.append(jax.ShapeDtypeStruct((2, 1024, 16), jnp.float32))
        scratch.append(pltpu.VMEM((_SUB, 16), jnp.float32))
        scratch.append(pltpu.VMEM_SHARED((1024, 16), jnp.float32))

    def body(*refs):
        if do_deg:
            (tabr, tabs, recv, send, ones_h, zer_h, zr, zs, degp,
             rawr, raws, idx2r, idx2s, bufr, bufs, sem0, sem1, onesv, dacc) = refs
        else:
            (tabr, tabs, recv, send, zr, zs,
             rawr, raws, idx2r, idx2s, bufr, bufs, sem0, sem1) = refs
        sems = (sem0, sem1)
        zrl = zr
        zsl = zs
        c = lax.axis_index("c")
        s = lax.axis_index("s")
        w = c * _NS + s
        e_base = w * _EPT
        pltpu.sync_copy(recv.at[pl.ds(e_base, _EPT)], rawr)
        pltpu.sync_copy(send.at[pl.ds(e_base, _EPT)], raws)
        if do_deg:
            pltpu.sync_copy(ones_h, onesv)
            pltpu.sync_copy(zer_h, dacc.at[pl.ds(s * 64, 64)])
            plsc.subcore_barrier()

        iters = [(b, ch) for b in range(B) for ch in range(_NCH)]

        def stage(i):
            b, ch = iters[i]
            sl = i % 2
            for j in range(_NSUB):
                for k in range(_SUB // 16):
                    src = pl.ds(ch * _CH + j * _SUB + k * 16, 16)
                    dst = pl.ds(k * 16, 16)
                    idx2r[sl, j, dst] = rawr[src] + b * N
                    idx2s[sl, j, dst] = raws[src] + b * N
            cps = []
            for j in range(_NSUB):
                d = pl.ds(j * _SUB, _SUB)
                cps.append(pltpu.async_copy(tabr.at[idx2r.at[sl, j]], bufr.at[sl, d], sems[sl]))
                cps.append(pltpu.async_copy(tabs.at[idx2s.at[sl, j]], bufs.at[sl, d], sems[sl]))
            return cps

        pend = stage(0)
        for i in range(len(iters)):
            b, ch = iters[i]
            sl = i % 2
            nxt_pend = stage(i + 1) if i + 1 < len(iters) else []
            for cp in pend:
                cp.wait()
            row0 = b * E + e_base + ch * _CH
            pltpu.sync_copy(bufr.at[sl], zrl.at[pl.ds(row0, _CH)])
            pltpu.sync_copy(bufs.at[sl], zsl.at[pl.ds(row0, _CH)])
            if do_deg and b == 0:
                for j in range(_NSUB):
                    pltpu.sync_copy(onesv, dacc.at[idx2r.at[sl, j]], add=True)
            pend = nxt_pend
        if do_deg:
            plsc.subcore_barrier()
            pltpu.sync_copy(dacc.at[pl.ds(s * 64, 64)], degp.at[c, pl.ds(s * 64, 64)])

    kw = dict(out_type=tuple(out_type), mesh=_sc_mesh,
              compiler_params=pltpu.CompilerParams(use_tc_tiling_on_sc=False),
              scratch_types=scratch)
    return functools.partial(pl.kernel, **kw)(body)


_sc_gather_deg = _make_sc_gather(True)
_sc_gather_nodeg = _make_sc_gather(False)


@functools.partial(
    pl.kernel,
    out_type=jax.ShapeDtypeStruct((2, 4096, H), jnp.float32),
    mesh=_sc_mesh,
    compiler_params=pltpu.CompilerParams(use_tc_tiling_on_sc=False),
    scratch_types=[
        pltpu.VMEM((_EPT,), jnp.int32),
        pltpu.VMEM((_EPT,), jnp.int32),
        pltpu.VMEM((2, _NSUB, _SUB), jnp.int32),
        pltpu.VMEM((2, _CH, H), jnp.float32),
        pltpu.VMEM_SHARED((4096, H), jnp.float32),
        pltpu.SemaphoreType.DMA,
        pltpu.SemaphoreType.DMA,
    ],
)
def _sc_scatter(m2, recv, mb, zer_h, pout, rawr, rawm, idx2, bufm, macc, sem0, sem1):
    sems = (sem0, sem1)
    m2l = m2
    c = lax.axis_index("c")
    s = lax.axis_index("s")
    w = c * _NS + s
    e_base = w * _EPT
    pltpu.sync_copy(recv.at[pl.ds(e_base, _EPT)], rawr)
    pltpu.sync_copy(zer_h, macc.at[pl.ds(s * 256, 256)])
    plsc.subcore_barrier()

    iters = [(b, ch) for b in range(B) for ch in range(_NCH)]

    pend = None
    for i in range(len(iters)):
        b, ch = iters[i]
        sl = i % 2
        row0 = b * E + e_base + ch * _CH
        if i == 0:
            pltpu.sync_copy(mb.at[pl.ds(row0, _CH)], rawm.at[pl.ds(0, _CH)])
            pend = pltpu.async_copy(m2l.at[pl.ds(row0, _CH)], bufm.at[sl], sems[sl])
            for j in range(_NSUB):
                for k in range(_SUB // 16):
                    msl = pl.ds(j * _SUB + k * 16, 16)
                    esl = pl.ds(ch * _CH + j * _SUB + k * 16, 16)
                    idx2[sl, j, pl.ds(k * 16, 16)] = (
                        rawm[msl] * (rawr[esl] - 1000) + (1000 + b * 1024))
        if i + 1 < len(iters):
            bn, chn = iters[i + 1]
            sln = (i + 1) % 2
            rown = bn * E + e_base + chn * _CH
            pltpu.sync_copy(mb.at[pl.ds(rown, _CH)], rawm.at[pl.ds(sln * _CH, _CH)])
            nxt = pltpu.async_copy(m2l.at[pl.ds(rown, _CH)], bufm.at[sln], sems[sln])
            for j in range(_NSUB):
                for k in range(_SUB // 16):
                    msl = pl.ds(sln * _CH + j * _SUB + k * 16, 16)
                    esl = pl.ds(chn * _CH + j * _SUB + k * 16, 16)
                    idx2[sln, j, pl.ds(k * 16, 16)] = (
                        rawm[msl] * (rawr[esl] - 1000) + (1000 + bn * 1024))
        else:
            nxt = None
        pend.wait()
        for j in range(_NSUB):
            pltpu.sync_copy(bufm.at[sl, pl.ds(j * _SUB, _SUB)], macc.at[idx2.at[sl, j]], add=True)
        pend = nxt
    plsc.subcore_barrier()
    pltpu.sync_copy(macc.at[pl.ds(s * 256, 256)], pout.at[c, pl.ds(s * 256, 256)])


def kernel(inputs, hidden, edge_logits, send_edges, recv_edges,
           msg_fc1_w, msg_fc1_b, msg_fc2_w, msg_fc2_b,
           hidden_r_w, hidden_i_w, hidden_h_w,
           input_r_w, input_r_b, input_i_w, input_i_b, input_n_w, input_n_b,
           out_w1, out_b1, out_w2, out_b2,
           proj_loc_w, proj_loc_b, proj_scale_w, proj_scale_b):
    f32 = jnp.float32

    g = jax.random.gumbel(jax.random.key(42), (BE, ET), dtype=f32)
    l0 = edge_logits[:, :, 0].reshape(2000, 128)
    l1 = edge_logits[:, :, 1].reshape(2000, 128)
    g0 = g[:, 0].reshape(2000, 128)
    g1 = g[:, 1].reshape(2000, 128)
    e0, e1, mbin = _edges_tc(l0, l1, g0, g1)
    edges = jnp.stack([e0.reshape(B, E), e1.reshape(B, E)], axis=-1)
    mb1d = mbin.reshape(BE)

    h2 = hidden.reshape(B * N, H)
    x2 = inputs.reshape(B * N, IN)
    w1 = msg_fc1_w[0, 1]
    hr0, hs0, ir, ii, inn = _node0_tc(
        h2, w1[:H], w1[H:], msg_fc1_b[0, 1].reshape(1, H), x2,
        input_r_w, input_r_b.reshape(1, H),
        input_i_w, input_i_b.reshape(1, H),
        input_n_w, input_n_b.reshape(1, H))

    ones16 = jnp.ones((_SUB, 16), f32)
    zer16 = jnp.zeros((64, 16), f32)
    zer64 = jnp.zeros((256, H), f32)

    def w2diag(w2, b2):
        wd = jnp.zeros((128, 128), f32)
        wd = wd.at[:H, :H].set(w2).at[H:, H:].set(w2)
        bd = jnp.concatenate([b2, b2]).reshape(1, 128)
        return wd, bd

    w2d0, b2d0 = w2diag(msg_fc2_w[0, 1], msg_fc2_b[0, 1])
    w2d1, b2d1 = w2diag(msg_fc2_w[1, 1], msg_fc2_b[1, 1])

    zr0, zs0, degp4 = _sc_gather_deg(hr0, hs0, recv_edges, send_edges, ones16, zer16)
    m2_0 = _fc2_tc(zr0.reshape(BEH, 128), zs0.reshape(BEH, 128), w2d0, b2d0)
    pout0 = _sc_scatter(m2_0.reshape(BE, H), recv_edges, mb1d, zer64)
    p0 = pout0.reshape(2, B, 1024, H)[:, :, :N, :].reshape(2, B * N, H)
    degp = degp4[:, :N, :]

    w1_1 = msg_fc1_w[1, 1]
    agg0, hr1, hs1, dcol = _comb_tc(
        p0[0], p0[1], degp, w1_1[:H], w1_1[H:], msg_fc1_b[1, 1].reshape(1, H))

    zr1, zs1 = _sc_gather_nodeg(hr1, hs1, recv_edges, send_edges)
    m2_1 = _fc2_tc(zr1.reshape(BEH, 128), zs1.reshape(BEH, 128), w2d1, b2d1)
    pout1 = _sc_scatter(m2_1.reshape(BE, H), recv_edges, mb1d, zer64)
    p1 = pout1.reshape(2, B, 1024, H)[:, :, :N, :].reshape(2, B * N, H)

    hn2, pred2 = _gru_tc(p1[0], p1[1], dcol, agg0, ir, ii, inn, h2,
                         hidden_r_w, hidden_i_w, hidden_h_w,
                         out_w1, out_b1.reshape(1, H), out_w2, out_b2.reshape(1, H))
    hidden_new = hn2.reshape(B, N, H)

    flatT = pred2.reshape(B, N * H).T
    locT, sclT = _proj_tc(proj_loc_w.T, proj_scale_w.T, flatT,
                          proj_loc_b.reshape(N, 1), proj_scale_b.reshape(N, 1))
    loc = locT.T
    scale = sclT.T

    return ((loc, scale), hidden_new, edges)


# speedup vs baseline: 11.8393x; 1.1215x over previous
import functools

import numpy as np

import jax
import jax.numpy as jnp
from jax import lax
from jax.experimental import pallas as pl
from jax.experimental.pallas import tpu as pltpu
from jax.experimental.pallas import tpu_sc as plsc

B = 4
N = 1000
E = 64000
H = 64
IN = 8
L = 2
ET = 2
TAU = 0.5

BE = B * E
BEH = BE // 2

def _gumbel_const(shape):
    n = int(np.prod(shape))
    with np.errstate(over="ignore"):
        k0 = np.uint32(0)
        k1 = np.uint32(42)
        ks = [k0, k1, np.uint32(k0 ^ k1 ^ np.uint32(0x1BD11BDA))]
        rot = [(13, 15, 26, 6), (17, 29, 16, 24)]

        def rounds(a, b, rots):
            for r in rots:
                a = (a + b).astype(np.uint32)
                b = ((b << np.uint32(r)) | (b >> np.uint32(32 - r))).astype(np.uint32)
                b = a ^ b
            return a, b

        a = np.full(n, ks[0], np.uint32)
        b = (np.arange(n, dtype=np.uint32) + ks[1]).astype(np.uint32)
        a, b = rounds(a, b, rot[0])
        a = (a + ks[1]).astype(np.uint32); b = (b + ks[2] + np.uint32(1)).astype(np.uint32)
        a, b = rounds(a, b, rot[1])
        a = (a + ks[2]).astype(np.uint32); b = (b + ks[0] + np.uint32(2)).astype(np.uint32)
        a, b = rounds(a, b, rot[0])
        a = (a + ks[0]).astype(np.uint32); b = (b + ks[1] + np.uint32(3)).astype(np.uint32)
        a, b = rounds(a, b, rot[1])
        a = (a + ks[1]).astype(np.uint32); b = (b + ks[2] + np.uint32(4)).astype(np.uint32)
        a, b = rounds(a, b, rot[0])
        a = (a + ks[2]).astype(np.uint32); b = (b + ks[0] + np.uint32(5)).astype(np.uint32)
        bits = (a ^ b).reshape(shape)
    fb = ((bits >> np.uint32(9)) | np.uint32(0x3F800000)).view(np.float32)
    f = (fb - np.float32(1.0)).astype(np.float32)
    tiny = np.float32(np.finfo(np.float32).tiny)
    mm = np.float32(np.float32(1.0) - tiny)
    u = np.maximum(tiny, (f * mm + tiny).astype(np.float32))
    return (-np.log(-np.log(u))).astype(np.float32)


_GUMBEL = _gumbel_const((BE, ET))


def _edges_body(l0, l1, g0, g1, e0, e1, mb):
    a = (l0[...] + g0[...]) / TAU
    b = (l1[...] + g1[...]) / TAU
    m = jnp.maximum(a, b)
    ea = jnp.exp(a - m)
    eb = jnp.exp(b - m)
    s = ea + eb
    y0 = ea / s
    y1 = eb / s
    hard1 = b > a
    h1 = hard1.astype(jnp.float32)
    h0 = 1.0 - h1
    e0[...] = (h0 - y0) + y0
    e1[...] = (h1 - y1) + y1
    mb[...] = hard1.astype(jnp.int32)


def _edges_tc(l0, l1, g0, g1):
    return pl.pallas_call(
        _edges_body,
        out_shape=(
            jax.ShapeDtypeStruct((2000, 128), jnp.float32),
            jax.ShapeDtypeStruct((2000, 128), jnp.float32),
            jax.ShapeDtypeStruct((2000, 128), jnp.int32),
        ),
    )(l0, l1, g0, g1)


def _node0_body(h2, w1a, w1b, b1, x2, wr, br, wi, bi, wn, bn,
                hr, hs, ir, ii, inn):
    h = h2[...]
    hr[...] = jnp.dot(h, w1a[...], preferred_element_type=jnp.float32) + b1[...]
    hs[...] = jnp.dot(h, w1b[...], preferred_element_type=jnp.float32)
    x = x2[...]
    ir[...] = jnp.dot(x, wr[...], preferred_element_type=jnp.float32) + br[...]
    ii[...] = jnp.dot(x, wi[...], preferred_element_type=jnp.float32) + bi[...]
    inn[...] = jnp.dot(x, wn[...], preferred_element_type=jnp.float32) + bn[...]


def _node0_tc(h2, w1a, w1b, b1, x2, wr, br, wi, bi, wn, bn):
    f = jax.ShapeDtypeStruct
    return pl.pallas_call(
        _node0_body,
        out_shape=(
            f((B * N, H), jnp.float32), f((B * N, H), jnp.float32),
            f((B * N, H), jnp.float32), f((B * N, H), jnp.float32),
            f((B * N, H), jnp.float32),
        ),
    )(h2, w1a, w1b, b1, x2, wr, br, wi, bi, wn, bn)


_FC2_BLK = 4000


def _fc2_body(zr, zs, w2d, b2d, out):
    m = jnp.tanh(zr[...] + zs[...])
    t = jnp.dot(m, w2d[...], preferred_element_type=jnp.float32) + b2d[...]
    out[...] = jnp.tanh(t)


def _fc2_tc(zr, zs, w2d, b2d):
    grid = BEH // _FC2_BLK
    return pl.pallas_call(
        _fc2_body,
        grid=(grid,),
        in_specs=[
            pl.BlockSpec((_FC2_BLK, 128), lambda i: (i, 0)),
            pl.BlockSpec((_FC2_BLK, 128), lambda i: (i, 0)),
            pl.BlockSpec((128, 128), lambda i: (0, 0)),
            pl.BlockSpec((1, 128), lambda i: (0, 0)),
        ],
        out_specs=pl.BlockSpec((_FC2_BLK, 128), lambda i: (i, 0)),
        out_shape=jax.ShapeDtypeStruct((BEH, 128), jnp.float32),
    )(zr, zs, w2d, b2d)


def _comb_body(p0, p1, dp, w1a, w1b, b1, agg0, hr, hs, dout):
    d = dp[0] + dp[1]
    dcol = d[:, 0:1]
    dcol = jnp.where(dcol == 0.0, 1.0, dcol)
    d4 = jnp.concatenate([dcol, dcol, dcol, dcol], axis=0)
    a0 = (p0[...] + p1[...]) / d4
    agg0[...] = a0
    hr[...] = jnp.dot(a0, w1a[...], preferred_element_type=jnp.float32) + b1[...]
    hs[...] = jnp.dot(a0, w1b[...], preferred_element_type=jnp.float32)
    dout[...] = dcol


def _comb_tc(p0, p1, dp, w1a, w1b, b1):
    f = jax.ShapeDtypeStruct
    return pl.pallas_call(
        _comb_body,
        out_shape=(
            f((B * N, H), jnp.float32), f((B * N, H), jnp.float32),
            f((B * N, H), jnp.float32), f((N, 1), jnp.float32),
        ),
    )(p0, p1, dp, w1a, w1b, b1)


def _gru_body(pa, pb, dcol, agg0, ir, ii, inn, h2, wr, wi, wh,
              ow1, ob1, ow2, ob2, hn_out, pred_out):
    dc = dcol[...]
    d4 = jnp.concatenate([dc, dc, dc, dc], axis=0)
    agg1 = (pa[...] + pb[...]) / d4
    am = jnp.concatenate([agg0[...], agg1], axis=1)
    r = jax.nn.sigmoid(ir[...] + jnp.dot(am, wr[...], preferred_element_type=jnp.float32))
    ig = jax.nn.sigmoid(ii[...] + jnp.dot(am, wi[...], preferred_element_type=jnp.float32))
    ng = jnp.tanh(inn[...] + r * jnp.dot(am, wh[...], preferred_element_type=jnp.float32))
    hn = (1.0 - ig) * ng + ig * h2[...]
    hn_out[...] = hn
    h1 = jax.nn.relu(jnp.dot(hn, ow1[...], preferred_element_type=jnp.float32) + ob1[...])
    pred_out[...] = jax.nn.relu(jnp.dot(h1, ow2[...], preferred_element_type=jnp.float32) + ob2[...])


def _gru_tc(pa, pb, dcol, agg0, ir, ii, inn, h2, wr, wi, wh, ow1, ob1, ow2, ob2):
    f = jax.ShapeDtypeStruct
    return pl.pallas_call(
        _gru_body,
        out_shape=(f((B * N, H), jnp.float32), f((B * N, H), jnp.float32)),
    )(pa, pb, dcol, agg0, ir, ii, inn, h2, wr, wi, wh, ow1, ob1, ow2, ob2)


_PROJ_KC = 1280


def _proj_body(wlt, wst, ft, bl, bs, loc, scl):
    i = pl.program_id(0)
    nsteps = pl.num_programs(0)
    fb = ft[...]
    pl_part = jnp.dot(wlt[...], fb, preferred_element_type=jnp.float32)
    ps_part = jnp.dot(wst[...], fb, preferred_element_type=jnp.float32)

    @pl.when(i == 0)
    def _():
        loc[...] = jnp.zeros_like(loc)
        scl[...] = jnp.zeros_like(scl)

    loc[...] += pl_part
    scl[...] += ps_part

    @pl.when(i == nsteps - 1)
    def _():
        loc[...] += bl[...]
        s = scl[...] + bs[...]
        scl[...] = jnp.log1p(jnp.exp(-jnp.abs(s))) + jax.nn.relu(s)


def _proj_tc(wlt, wst, ft, bl, bs):
    grid = (N * H) // _PROJ_KC
    f = jax.ShapeDtypeStruct
    return pl.pallas_call(
        _proj_body,
        grid=(grid,),
        in_specs=[
            pl.BlockSpec((N, _PROJ_KC), lambda i: (0, i)),
            pl.BlockSpec((N, _PROJ_KC), lambda i: (0, i)),
            pl.BlockSpec((_PROJ_KC, B), lambda i: (i, 0)),
            pl.BlockSpec((N, 1), lambda i: (0, 0)),
            pl.BlockSpec((N, 1), lambda i: (0, 0)),
        ],
        out_specs=(
            pl.BlockSpec((N, B), lambda i: (0, 0)),
            pl.BlockSpec((N, B), lambda i: (0, 0)),
        ),
        out_shape=(f((N, B), jnp.float32), f((N, B), jnp.float32)),
    )(wlt, wst, ft, bl, bs)


_NC = 2
_NS = 16
_NW = _NC * _NS
_EPT = E // _NW
_CH = 400
_NCH = _EPT // _CH
_SUB = 80
_NSUB = _CH // _SUB

_sc_mesh = plsc.VectorSubcoreMesh(core_axis_name="c", subcore_axis_name="s")


def _make_sc_gather(do_deg):
    out_type = [
        jax.ShapeDtypeStruct((BE, H), jnp.float32),
        jax.ShapeDtypeStruct((BE, H), jnp.float32),
    ]
    scratch = [
        pltpu.VMEM((_EPT,), jnp.int32),
        pltpu.VMEM((_EPT,), jnp.int32),
        pltpu.VMEM((2, _NSUB, _SUB), jnp.int32),
        pltpu.VMEM((2, _NSUB, _SUB), jnp.int32),
        pltpu.VMEM((2, _CH, H), jnp.float32),
        pltpu.VMEM((2, _CH, H), jnp.float32),
        pltpu.SemaphoreType.DMA,
        pltpu.SemaphoreType.DMA,
    ]
    if do_deg:
        out_type.append(jax.ShapeDtypeStruct((2, 1024, 16), jnp.float32))
        scratch.append(pltpu.VMEM((_SUB, 16), jnp.float32))
        scratch.append(pltpu.VMEM_SHARED((1024, 16), jnp.float32))

    def body(*refs):
        if do_deg:
            (tabr, tabs, recv, send, ones_h, zer_h, zr, zs, degp,
             rawr, raws, idx2r, idx2s, bufr, bufs, sem0, sem1, onesv, dacc) = refs
        else:
            (tabr, tabs, recv, send, zr, zs,
             rawr, raws, idx2r, idx2s, bufr, bufs, sem0, sem1) = refs
        sems = (sem0, sem1)
        zrl = zr
        zsl = zs
        c = lax.axis_index("c")
        s = lax.axis_index("s")
        w = c * _NS + s
        e_base = w * _EPT
        pltpu.sync_copy(recv.at[pl.ds(e_base, _EPT)], rawr)
        pltpu.sync_copy(send.at[pl.ds(e_base, _EPT)], raws)
        if do_deg:
            pltpu.sync_copy(ones_h, onesv)
            pltpu.sync_copy(zer_h, dacc.at[pl.ds(s * 64, 64)])
            plsc.subcore_barrier()

        iters = [(b, ch) for b in range(B) for ch in range(_NCH)]

        def stage(i):
            b, ch = iters[i]
            sl = i % 2
            for j in range(_NSUB):
                for k in range(_SUB // 16):
                    src = pl.ds(ch * _CH + j * _SUB + k * 16, 16)
                    dst = pl.ds(k * 16, 16)
                    idx2r[sl, j, dst] = rawr[src] + b * N
                    idx2s[sl, j, dst] = raws[src] + b * N
            cps = []
            for j in range(_NSUB):
                d = pl.ds(j * _SUB, _SUB)
                cps.append(pltpu.async_copy(tabr.at[idx2r.at[sl, j]], bufr.at[sl, d], sems[sl]))
                cps.append(pltpu.async_copy(tabs.at[idx2s.at[sl, j]], bufs.at[sl, d], sems[sl]))
            return cps

        pend = stage(0)
        for i in range(len(iters)):
            b, ch = iters[i]
            sl = i % 2
            nxt_pend = stage(i + 1) if i + 1 < len(iters) else []
            for cp in pend:
                cp.wait()
            row0 = b * E + e_base + ch * _CH
            pltpu.sync_copy(bufr.at[sl], zrl.at[pl.ds(row0, _CH)])
            pltpu.sync_copy(bufs.at[sl], zsl.at[pl.ds(row0, _CH)])
            if do_deg and b == 0:
                for j in range(_NSUB):
                    pltpu.sync_copy(onesv, dacc.at[idx2r.at[sl, j]], add=True)
            pend = nxt_pend
        if do_deg:
            plsc.subcore_barrier()
            pltpu.sync_copy(dacc.at[pl.ds(s * 64, 64)], degp.at[c, pl.ds(s * 64, 64)])

    kw = dict(out_type=tuple(out_type), mesh=_sc_mesh,
              compiler_params=pltpu.CompilerParams(use_tc_tiling_on_sc=False),
              scratch_types=scratch)
    return functools.partial(pl.kernel, **kw)(body)


_sc_gather_deg = _make_sc_gather(True)
_sc_gather_nodeg = _make_sc_gather(False)


@functools.partial(
    pl.kernel,
    out_type=jax.ShapeDtypeStruct((2, 4096, H), jnp.float32),
    mesh=_sc_mesh,
    compiler_params=pltpu.CompilerParams(use_tc_tiling_on_sc=False),
    scratch_types=[
        pltpu.VMEM((_EPT,), jnp.int32),
        pltpu.VMEM((_EPT,), jnp.int32),
        pltpu.VMEM((2, _NSUB, _SUB), jnp.int32),
        pltpu.VMEM((2, _CH, H), jnp.float32),
        pltpu.VMEM_SHARED((4096, H), jnp.float32),
        pltpu.SemaphoreType.DMA,
        pltpu.SemaphoreType.DMA,
    ],
)
def _sc_scatter(m2, recv, mb, zer_h, pout, rawr, rawm, idx2, bufm, macc, sem0, sem1):
    sems = (sem0, sem1)
    m2l = m2
    c = lax.axis_index("c")
    s = lax.axis_index("s")
    w = c * _NS + s
    e_base = w * _EPT
    pltpu.sync_copy(recv.at[pl.ds(e_base, _EPT)], rawr)
    pltpu.sync_copy(zer_h, macc.at[pl.ds(s * 256, 256)])
    plsc.subcore_barrier()

    iters = [(b, ch) for b in range(B) for ch in range(_NCH)]

    pend = None
    for i in range(len(iters)):
        b, ch = iters[i]
        sl = i % 2
        row0 = b * E + e_base + ch * _CH
        if i == 0:
            pltpu.sync_copy(mb.at[pl.ds(row0, _CH)], rawm.at[pl.ds(0, _CH)])
            pend = pltpu.async_copy(m2l.at[pl.ds(row0, _CH)], bufm.at[sl], sems[sl])
            for j in range(_NSUB):
                for k in range(_SUB // 16):
                    msl = pl.ds(j * _SUB + k * 16, 16)
                    esl = pl.ds(ch * _CH + j * _SUB + k * 16, 16)
                    idx2[sl, j, pl.ds(k * 16, 16)] = (
                        rawm[msl] * (rawr[esl] - 1000) + (1000 + b * 1024))
        if i + 1 < len(iters):
            bn, chn = iters[i + 1]
            sln = (i + 1) % 2
            rown = bn * E + e_base + chn * _CH
            pltpu.sync_copy(mb.at[pl.ds(rown, _CH)], rawm.at[pl.ds(sln * _CH, _CH)])
            nxt = pltpu.async_copy(m2l.at[pl.ds(rown, _CH)], bufm.at[sln], sems[sln])
            for j in range(_NSUB):
                for k in range(_SUB // 16):
                    msl = pl.ds(sln * _CH + j * _SUB + k * 16, 16)
                    esl = pl.ds(chn * _CH + j * _SUB + k * 16, 16)
                    idx2[sln, j, pl.ds(k * 16, 16)] = (
                        rawm[msl] * (rawr[esl] - 1000) + (1000 + bn * 1024))
        else:
            nxt = None
        pend.wait()
        for j in range(_NSUB):
            pltpu.sync_copy(bufm.at[sl, pl.ds(j * _SUB, _SUB)], macc.at[idx2.at[sl, j]], add=True)
        pend = nxt
    plsc.subcore_barrier()
    pltpu.sync_copy(macc.at[pl.ds(s * 256, 256)], pout.at[c, pl.ds(s * 256, 256)])


def kernel(inputs, hidden, edge_logits, send_edges, recv_edges,
           msg_fc1_w, msg_fc1_b, msg_fc2_w, msg_fc2_b,
           hidden_r_w, hidden_i_w, hidden_h_w,
           input_r_w, input_r_b, input_i_w, input_i_b, input_n_w, input_n_b,
           out_w1, out_b1, out_w2, out_b2,
           proj_loc_w, proj_loc_b, proj_scale_w, proj_scale_b):
    f32 = jnp.float32

    g = _GUMBEL
    l0 = edge_logits[:, :, 0].reshape(2000, 128)
    l1 = edge_logits[:, :, 1].reshape(2000, 128)
    g0 = jnp.asarray(g[:, 0].reshape(2000, 128))
    g1 = jnp.asarray(g[:, 1].reshape(2000, 128))
    e0, e1, mbin = _edges_tc(l0, l1, g0, g1)
    edges = jnp.stack([e0.reshape(B, E), e1.reshape(B, E)], axis=-1)
    mb1d = mbin.reshape(BE)

    h2 = hidden.reshape(B * N, H)
    x2 = inputs.reshape(B * N, IN)
    w1 = msg_fc1_w[0, 1]
    hr0, hs0, ir, ii, inn = _node0_tc(
        h2, w1[:H], w1[H:], msg_fc1_b[0, 1].reshape(1, H), x2,
        input_r_w, input_r_b.reshape(1, H),
        input_i_w, input_i_b.reshape(1, H),
        input_n_w, input_n_b.reshape(1, H))

    ones16 = jnp.ones((_SUB, 16), f32)
    zer16 = jnp.zeros((64, 16), f32)
    zer64 = jnp.zeros((256, H), f32)

    def w2diag(w2, b2):
        wd = jnp.zeros((128, 128), f32)
        wd = wd.at[:H, :H].set(w2).at[H:, H:].set(w2)
        bd = jnp.concatenate([b2, b2]).reshape(1, 128)
        return wd, bd

    w2d0, b2d0 = w2diag(msg_fc2_w[0, 1], msg_fc2_b[0, 1])
    w2d1, b2d1 = w2diag(msg_fc2_w[1, 1], msg_fc2_b[1, 1])

    zr0, zs0, degp4 = _sc_gather_deg(hr0, hs0, recv_edges, send_edges, ones16, zer16)
    m2_0 = _fc2_tc(zr0.reshape(BEH, 128), zs0.reshape(BEH, 128), w2d0, b2d0)
    pout0 = _sc_scatter(m2_0.reshape(BE, H), recv_edges, mb1d, zer64)
    p0 = pout0.reshape(2, B, 1024, H)[:, :, :N, :].reshape(2, B * N, H)
    degp = degp4[:, :N, :]

    w1_1 = msg_fc1_w[1, 1]
    agg0, hr1, hs1, dcol = _comb_tc(
        p0[0], p0[1], degp, w1_1[:H], w1_1[H:], msg_fc1_b[1, 1].reshape(1, H))

    zr1, zs1 = _sc_gather_nodeg(hr1, hs1, recv_edges, send_edges)
    m2_1 = _fc2_tc(zr1.reshape(BEH, 128), zs1.reshape(BEH, 128), w2d1, b2d1)
    pout1 = _sc_scatter(m2_1.reshape(BE, H), recv_edges, mb1d, zer64)
    p1 = pout1.reshape(2, B, 1024, H)[:, :, :N, :].reshape(2, B * N, H)

    hn2, pred2 = _gru_tc(p1[0], p1[1], dcol, agg0, ir, ii, inn, h2,
                         hidden_r_w, hidden_i_w, hidden_h_w,
                         out_w1, out_b1.reshape(1, H), out_w2, out_b2.reshape(1, H))
    hidden_new = hn2.reshape(B, N, H)

    flatT = pred2.reshape(B, N * H).T
    locT, sclT = _proj_tc(proj_loc_w.T, proj_scale_w.T, flatT,
                          proj_loc_b.reshape(N, 1), proj_scale_b.reshape(N, 1))
    loc = locT.T
    scale = sclT.T

    return ((loc, scale), hidden_new, edges)


# speedup vs baseline: 12.3025x; 1.0391x over previous
import functools

import numpy as np

import jax
import jax.numpy as jnp
from jax import lax
from jax.experimental import pallas as pl
from jax.experimental.pallas import tpu as pltpu
from jax.experimental.pallas import tpu_sc as plsc

B = 4
N = 1000
E = 64000
H = 64
IN = 8
L = 2
ET = 2
TAU = 0.5

BE = B * E
BEH = BE // 2

def _gumbel_const(shape):
    n = int(np.prod(shape))
    with np.errstate(over="ignore"):
        k0 = np.uint32(0)
        k1 = np.uint32(42)
        ks = [k0, k1, np.uint32(k0 ^ k1 ^ np.uint32(0x1BD11BDA))]
        rot = [(13, 15, 26, 6), (17, 29, 16, 24)]

        def rounds(a, b, rots):
            for r in rots:
                a = (a + b).astype(np.uint32)
                b = ((b << np.uint32(r)) | (b >> np.uint32(32 - r))).astype(np.uint32)
                b = a ^ b
            return a, b

        a = np.full(n, ks[0], np.uint32)
        b = (np.arange(n, dtype=np.uint32) + ks[1]).astype(np.uint32)
        a, b = rounds(a, b, rot[0])
        a = (a + ks[1]).astype(np.uint32); b = (b + ks[2] + np.uint32(1)).astype(np.uint32)
        a, b = rounds(a, b, rot[1])
        a = (a + ks[2]).astype(np.uint32); b = (b + ks[0] + np.uint32(2)).astype(np.uint32)
        a, b = rounds(a, b, rot[0])
        a = (a + ks[0]).astype(np.uint32); b = (b + ks[1] + np.uint32(3)).astype(np.uint32)
        a, b = rounds(a, b, rot[1])
        a = (a + ks[1]).astype(np.uint32); b = (b + ks[2] + np.uint32(4)).astype(np.uint32)
        a, b = rounds(a, b, rot[0])
        a = (a + ks[2]).astype(np.uint32); b = (b + ks[0] + np.uint32(5)).astype(np.uint32)
        bits = (a ^ b).reshape(shape)
    fb = ((bits >> np.uint32(9)) | np.uint32(0x3F800000)).view(np.float32)
    f = (fb - np.float32(1.0)).astype(np.float32)
    tiny = np.float32(np.finfo(np.float32).tiny)
    mm = np.float32(np.float32(1.0) - tiny)
    u = np.maximum(tiny, (f * mm + tiny).astype(np.float32))
    return (-np.log(-np.log(u))).astype(np.float32)


_GUMBEL = _gumbel_const((BE, ET))


def _edges_body(l0, l1, g0, g1, e0, e1, mb):
    a = (l0[...] + g0[...]) / TAU
    b = (l1[...] + g1[...]) / TAU
    m = jnp.maximum(a, b)
    ea = jnp.exp(a - m)
    eb = jnp.exp(b - m)
    s = ea + eb
    y0 = ea / s
    y1 = eb / s
    hard1 = b > a
    h1 = hard1.astype(jnp.float32)
    h0 = 1.0 - h1
    e0[...] = (h0 - y0) + y0
    e1[...] = (h1 - y1) + y1
    mb[...] = hard1.astype(jnp.int32)


def _edges_tc(l0, l1, g0, g1):
    return pl.pallas_call(
        _edges_body,
        out_shape=(
            jax.ShapeDtypeStruct((2000, 128), jnp.float32),
            jax.ShapeDtypeStruct((2000, 128), jnp.float32),
            jax.ShapeDtypeStruct((2000, 128), jnp.int32),
        ),
    )(l0, l1, g0, g1)


def _node0_body(h2, w1a, w1b, b1, x2, wr, br, wi, bi, wn, bn,
                hr, hs, ir, ii, inn):
    h = h2[...]
    hr[...] = jnp.dot(h, w1a[...], preferred_element_type=jnp.float32) + b1[...]
    hs[...] = jnp.dot(h, w1b[...], preferred_element_type=jnp.float32)
    x = x2[...]
    ir[...] = jnp.dot(x, wr[...], preferred_element_type=jnp.float32) + br[...]
    ii[...] = jnp.dot(x, wi[...], preferred_element_type=jnp.float32) + bi[...]
    inn[...] = jnp.dot(x, wn[...], preferred_element_type=jnp.float32) + bn[...]


def _node0_tc(h2, w1a, w1b, b1, x2, wr, br, wi, bi, wn, bn):
    f = jax.ShapeDtypeStruct
    return pl.pallas_call(
        _node0_body,
        out_shape=(
            f((B * N, H), jnp.float32), f((B * N, H), jnp.float32),
            f((B * N, H), jnp.float32), f((B * N, H), jnp.float32),
            f((B * N, H), jnp.float32),
        ),
    )(h2, w1a, w1b, b1, x2, wr, br, wi, bi, wn, bn)


_FC2_BLK = 4000


def _fc2_body(zr, zs, w2d, b2d, out):
    m = jnp.tanh(zr[...] + zs[...])
    t = jnp.dot(m, w2d[...], preferred_element_type=jnp.float32) + b2d[...]
    out[...] = jnp.tanh(t)


def _fc2_tc(zr, zs, w2d, b2d):
    grid = zr.shape[0] // _FC2_BLK
    return pl.pallas_call(
        _fc2_body,
        grid=(grid,),
        in_specs=[
            pl.BlockSpec((_FC2_BLK, 128), lambda i: (i, 0)),
            pl.BlockSpec((_FC2_BLK, 128), lambda i: (i, 0)),
            pl.BlockSpec((128, 128), lambda i: (0, 0)),
            pl.BlockSpec((1, 128), lambda i: (0, 0)),
        ],
        out_specs=pl.BlockSpec((_FC2_BLK, 128), lambda i: (i, 0)),
        out_shape=jax.ShapeDtypeStruct(zr.shape, jnp.float32),
    )(zr, zs, w2d, b2d)


def _comb_body(pa, pb, dp, w1a, w1b, b1, agg0, hr, hs, dout):
    d = dp[0] + dp[1]
    dcol = d[:, 0:1]
    dcol = jnp.where(dcol == 0.0, 1.0, dcol)
    d4 = jnp.concatenate([dcol, dcol, dcol, dcol], axis=0)
    a0 = jnp.concatenate([pa[0] + pa[1], pb[0] + pb[1]], axis=0) / d4
    agg0[...] = a0
    hr[...] = jnp.dot(a0, w1a[...], preferred_element_type=jnp.float32) + b1[...]
    hs[...] = jnp.dot(a0, w1b[...], preferred_element_type=jnp.float32)
    dout[...] = dcol


def _comb_tc(pa, pb, dp, w1a, w1b, b1):
    f = jax.ShapeDtypeStruct
    return pl.pallas_call(
        _comb_body,
        out_shape=(
            f((B * N, H), jnp.float32), f((B * N, H), jnp.float32),
            f((B * N, H), jnp.float32), f((N, 1), jnp.float32),
        ),
    )(pa, pb, dp, w1a, w1b, b1)


def _gru_body(pa, pb, dcol, agg0, ir, ii, inn, h2, wr, wi, wh,
              ow1, ob1, ow2, ob2, hn_out, pred_out):
    dc = dcol[...]
    d4 = jnp.concatenate([dc, dc, dc, dc], axis=0)
    agg1 = jnp.concatenate([pa[0] + pa[1], pb[0] + pb[1]], axis=0) / d4
    am = jnp.concatenate([agg0[...], agg1], axis=1)
    r = jax.nn.sigmoid(ir[...] + jnp.dot(am, wr[...], preferred_element_type=jnp.float32))
    ig = jax.nn.sigmoid(ii[...] + jnp.dot(am, wi[...], preferred_element_type=jnp.float32))
    ng = jnp.tanh(inn[...] + r * jnp.dot(am, wh[...], preferred_element_type=jnp.float32))
    hn = (1.0 - ig) * ng + ig * h2[...]
    hn_out[...] = hn
    h1 = jax.nn.relu(jnp.dot(hn, ow1[...], preferred_element_type=jnp.float32) + ob1[...])
    pred_out[...] = jax.nn.relu(jnp.dot(h1, ow2[...], preferred_element_type=jnp.float32) + ob2[...])


def _gru_tc(pa, pb, dcol, agg0, ir, ii, inn, h2, wr, wi, wh, ow1, ob1, ow2, ob2):
    f = jax.ShapeDtypeStruct
    return pl.pallas_call(
        _gru_body,
        out_shape=(f((B * N, H), jnp.float32), f((B * N, H), jnp.float32)),
    )(pa, pb, dcol, agg0, ir, ii, inn, h2, wr, wi, wh, ow1, ob1, ow2, ob2)


_PROJ_KC = 1280


def _proj_body(wlt, wst, ft, bl, bs, loc, scl):
    i = pl.program_id(0)
    nsteps = pl.num_programs(0)
    fb = ft[...]
    pl_part = jnp.dot(wlt[...], fb, preferred_element_type=jnp.float32)
    ps_part = jnp.dot(wst[...], fb, preferred_element_type=jnp.float32)

    @pl.when(i == 0)
    def _():
        loc[...] = jnp.zeros_like(loc)
        scl[...] = jnp.zeros_like(scl)

    loc[...] += pl_part
    scl[...] += ps_part

    @pl.when(i == nsteps - 1)
    def _():
        loc[...] += bl[...]
        s = scl[...] + bs[...]
        scl[...] = jnp.log1p(jnp.exp(-jnp.abs(s))) + jax.nn.relu(s)


def _proj_tc(wlt, wst, ft, bl, bs):
    grid = (N * H) // _PROJ_KC
    f = jax.ShapeDtypeStruct
    return pl.pallas_call(
        _proj_body,
        grid=(grid,),
        in_specs=[
            pl.BlockSpec((N, _PROJ_KC), lambda i: (0, i)),
            pl.BlockSpec((N, _PROJ_KC), lambda i: (0, i)),
            pl.BlockSpec((_PROJ_KC, B), lambda i: (i, 0)),
            pl.BlockSpec((N, 1), lambda i: (0, 0)),
            pl.BlockSpec((N, 1), lambda i: (0, 0)),
        ],
        out_specs=(
            pl.BlockSpec((N, B), lambda i: (0, 0)),
            pl.BlockSpec((N, B), lambda i: (0, 0)),
        ),
        out_shape=(f((N, B), jnp.float32), f((N, B), jnp.float32)),
    )(wlt, wst, ft, bl, bs)


_NC = 2
_NS = 16
_NW = _NC * _NS
_EPT = E // _NW
_CH = 400
_NCH = _EPT // _CH
_SUB = 80
_NSUB = _CH // _SUB

_sc_mesh = plsc.VectorSubcoreMesh(core_axis_name="c", subcore_axis_name="s")


def _make_sc_gather(do_deg, b_lo):
    out_type = [
        jax.ShapeDtypeStruct((2 * E, H), jnp.float32),
        jax.ShapeDtypeStruct((2 * E, H), jnp.float32),
    ]
    scratch = [
        pltpu.VMEM((_EPT,), jnp.int32),
        pltpu.VMEM((_EPT,), jnp.int32),
        pltpu.VMEM((2, _NSUB, _SUB), jnp.int32),
        pltpu.VMEM((2, _NSUB, _SUB), jnp.int32),
        pltpu.VMEM((2, _CH, H), jnp.float32),
        pltpu.VMEM((2, _CH, H), jnp.float32),
        pltpu.SemaphoreType.DMA,
        pltpu.SemaphoreType.DMA,
    ]
    if do_deg:
        out_type.append(jax.ShapeDtypeStruct((2, 1024, 16), jnp.float32))
        scratch.append(pltpu.VMEM((_SUB, 16), jnp.float32))
        scratch.append(pltpu.VMEM_SHARED((1024, 16), jnp.float32))

    def body(*refs):
        if do_deg:
            (tabr, tabs, recv, send, ones_h, zer_h, zr, zs, degp,
             rawr, raws, idx2r, idx2s, bufr, bufs, sem0, sem1, onesv, dacc) = refs
        else:
            (tabr, tabs, recv, send, zr, zs,
             rawr, raws, idx2r, idx2s, bufr, bufs, sem0, sem1) = refs
        sems = (sem0, sem1)
        c = lax.axis_index("c")
        s = lax.axis_index("s")
        w = c * _NS + s
        e_base = w * _EPT
        pltpu.sync_copy(recv.at[pl.ds(e_base, _EPT)], rawr)
        pltpu.sync_copy(send.at[pl.ds(e_base, _EPT)], raws)
        if do_deg:
            pltpu.sync_copy(ones_h, onesv)
            pltpu.sync_copy(zer_h, dacc.at[pl.ds(s * 64, 64)])
            plsc.subcore_barrier()

        iters = [(bi, ch) for bi in range(2) for ch in range(_NCH)]

        def stage(i):
            bi, ch = iters[i]
            sl = i % 2
            for j in range(_NSUB):
                for k in range(_SUB // 16):
                    srcs = pl.ds(ch * _CH + j * _SUB + k * 16, 16)
                    dst = pl.ds(k * 16, 16)
                    idx2r[sl, j, dst] = rawr[srcs] + (b_lo + bi) * N
                    idx2s[sl, j, dst] = raws[srcs] + (b_lo + bi) * N
            cps = []
            for j in range(_NSUB):
                d = pl.ds(j * _SUB, _SUB)
                cps.append(pltpu.async_copy(tabr.at[idx2r.at[sl, j]], bufr.at[sl, d], sems[sl]))
                cps.append(pltpu.async_copy(tabs.at[idx2s.at[sl, j]], bufs.at[sl, d], sems[sl]))
            return cps

        pend = stage(0)
        for i in range(len(iters)):
            bi, ch = iters[i]
            sl = i % 2
            nxt_pend = stage(i + 1) if i + 1 < len(iters) else []
            for cp in pend:
                cp.wait()
            row0 = bi * E + e_base + ch * _CH
            pltpu.sync_copy(bufr.at[sl], zr.at[pl.ds(row0, _CH)])
            pltpu.sync_copy(bufs.at[sl], zs.at[pl.ds(row0, _CH)])
            if do_deg and b_lo + bi == 0:
                for j in range(_NSUB):
                    pltpu.sync_copy(onesv, dacc.at[idx2r.at[sl, j]], add=True)
            pend = nxt_pend
        if do_deg:
            plsc.subcore_barrier()
            pltpu.sync_copy(dacc.at[pl.ds(s * 64, 64)], degp.at[c, pl.ds(s * 64, 64)])

    kw = dict(out_type=tuple(out_type), mesh=_sc_mesh,
              compiler_params=pltpu.CompilerParams(use_tc_tiling_on_sc=False),
              scratch_types=scratch)
    return functools.partial(pl.kernel, **kw)(body)


_sc_gather_a = _make_sc_gather(True, 0)
_sc_gather_b = _make_sc_gather(False, 2)
_sc_gather_a1 = _make_sc_gather(False, 0)


def _make_sc_scatter(b_lo):
    def body(m2, recv, mb, zer_h, pout, rawr, rawm, idx2, bufm, macc, sem0, sem1):
        sems = (sem0, sem1)
        c = lax.axis_index("c")
        s = lax.axis_index("s")
        w = c * _NS + s
        e_base = w * _EPT
        pltpu.sync_copy(recv.at[pl.ds(e_base, _EPT)], rawr)
        pltpu.sync_copy(zer_h, macc.at[pl.ds(s * 128, 128)])
        plsc.subcore_barrier()

        iters = [(bi, ch) for bi in range(2) for ch in range(_NCH)]

        def stage(i):
            bi, ch = iters[i]
            sl = i % 2
            rloc = bi * E + e_base + ch * _CH
            rglob = (b_lo + bi) * E + e_base + ch * _CH
            pltpu.sync_copy(mb.at[pl.ds(rglob, _CH)], rawm.at[pl.ds(sl * _CH, _CH)])
            cp = pltpu.async_copy(m2.at[pl.ds(rloc, _CH)], bufm.at[sl], sems[sl])
            for j in range(_NSUB):
                for k in range(_SUB // 16):
                    msl = pl.ds(sl * _CH + j * _SUB + k * 16, 16)
                    esl = pl.ds(ch * _CH + j * _SUB + k * 16, 16)
                    idx2[sl, j, pl.ds(k * 16, 16)] = (
                        rawm[msl] * (rawr[esl] - 1000) + (1000 + bi * 1024))
            return cp

        pend = stage(0)
        for i in range(len(iters)):
            sl = i % 2
            nxt = stage(i + 1) if i + 1 < len(iters) else None
            pend.wait()
            for j in range(_NSUB):
                pltpu.sync_copy(bufm.at[sl, pl.ds(j * _SUB, _SUB)], macc.at[idx2.at[sl, j]], add=True)
            pend = nxt
        plsc.subcore_barrier()
        pltpu.sync_copy(macc.at[pl.ds(s * 128, 128)], pout.at[c, pl.ds(s * 128, 128)])

    return functools.partial(
        pl.kernel,
        out_type=jax.ShapeDtypeStruct((2, 2048, H), jnp.float32),
        mesh=_sc_mesh,
        compiler_params=pltpu.CompilerParams(use_tc_tiling_on_sc=False),
        scratch_types=[
            pltpu.VMEM((_EPT,), jnp.int32),
            pltpu.VMEM((2 * _CH,), jnp.int32),
            pltpu.VMEM((2, _NSUB, _SUB), jnp.int32),
            pltpu.VMEM((2, _CH, H), jnp.float32),
            pltpu.VMEM_SHARED((2048, H), jnp.float32),
            pltpu.SemaphoreType.DMA,
            pltpu.SemaphoreType.DMA,
        ],
    )(body)


_sc_scatter_a = _make_sc_scatter(0)
_sc_scatter_b = _make_sc_scatter(2)


def kernel(inputs, hidden, edge_logits, send_edges, recv_edges,
           msg_fc1_w, msg_fc1_b, msg_fc2_w, msg_fc2_b,
           hidden_r_w, hidden_i_w, hidden_h_w,
           input_r_w, input_r_b, input_i_w, input_i_b, input_n_w, input_n_b,
           out_w1, out_b1, out_w2, out_b2,
           proj_loc_w, proj_loc_b, proj_scale_w, proj_scale_b):
    f32 = jnp.float32

    g = _GUMBEL
    l0 = edge_logits[:, :, 0].reshape(2000, 128)
    l1 = edge_logits[:, :, 1].reshape(2000, 128)
    g0 = jnp.asarray(g[:, 0].reshape(2000, 128))
    g1 = jnp.asarray(g[:, 1].reshape(2000, 128))
    e0, e1, mbin = _edges_tc(l0, l1, g0, g1)
    edges = jnp.stack([e0.reshape(B, E), e1.reshape(B, E)], axis=-1)
    mb1d = mbin.reshape(BE)

    h2 = hidden.reshape(B * N, H)
    x2 = inputs.reshape(B * N, IN)
    w1 = msg_fc1_w[0, 1]
    hr0, hs0, ir, ii, inn = _node0_tc(
        h2, w1[:H], w1[H:], msg_fc1_b[0, 1].reshape(1, H), x2,
        input_r_w, input_r_b.reshape(1, H),
        input_i_w, input_i_b.reshape(1, H),
        input_n_w, input_n_b.reshape(1, H))

    ones16 = jnp.ones((_SUB, 16), f32)
    zer16 = jnp.zeros((64, 16), f32)
    zer64 = jnp.zeros((128, H), f32)

    def w2diag(w2, b2):
        wd = jnp.zeros((128, 128), f32)
        wd = wd.at[:H, :H].set(w2).at[H:, H:].set(w2)
        bd = jnp.concatenate([b2, b2]).reshape(1, 128)
        return wd, bd

    w2d0, b2d0 = w2diag(msg_fc2_w[0, 1], msg_fc2_b[0, 1])
    w2d1, b2d1 = w2diag(msg_fc2_w[1, 1], msg_fc2_b[1, 1])

    zrA, zsA, degp4 = _sc_gather_a(hr0, hs0, recv_edges, send_edges, ones16, zer16)
    zrB, zsB = _sc_gather_b(hr0, hs0, recv_edges, send_edges)
    m2A = _fc2_tc(zrA.reshape(E, 128), zsA.reshape(E, 128), w2d0, b2d0)
    m2B = _fc2_tc(zrB.reshape(E, 128), zsB.reshape(E, 128), w2d0, b2d0)
    pA = _sc_scatter_a(m2A.reshape(2 * E, H), recv_edges, mb1d, zer64)
    pB = _sc_scatter_b(m2B.reshape(2 * E, H), recv_edges, mb1d, zer64)
    pa0 = pA.reshape(2, 2, 1024, H)[:, :, :N, :].reshape(2, 2 * N, H)
    pb0 = pB.reshape(2, 2, 1024, H)[:, :, :N, :].reshape(2, 2 * N, H)
    degp = degp4[:, :N, :]

    w1_1 = msg_fc1_w[1, 1]
    agg0, hr1, hs1, dcol = _comb_tc(
        pa0, pb0, degp, w1_1[:H], w1_1[H:], msg_fc1_b[1, 1].reshape(1, H))

    zrA1, zsA1 = _sc_gather_a1(hr1, hs1, recv_edges, send_edges)
    zrB1, zsB1 = _sc_gather_b(hr1, hs1, recv_edges, send_edges)
    m2A1 = _fc2_tc(zrA1.reshape(E, 128), zsA1.reshape(E, 128), w2d1, b2d1)
    m2B1 = _fc2_tc(zrB1.reshape(E, 128), zsB1.reshape(E, 128), w2d1, b2d1)
    pA1 = _sc_scatter_a(m2A1.reshape(2 * E, H), recv_edges, mb1d, zer64)
    pB1 = _sc_scatter_b(m2B1.reshape(2 * E, H), recv_edges, mb1d, zer64)
    pa1 = pA1.reshape(2, 2, 1024, H)[:, :, :N, :].reshape(2, 2 * N, H)
    pb1 = pB1.reshape(2, 2, 1024, H)[:, :, :N, :].reshape(2, 2 * N, H)

    hn2, pred2 = _gru_tc(pa1, pb1, dcol, agg0, ir, ii, inn, h2,
                         hidden_r_w, hidden_i_w, hidden_h_w,
                         out_w1, out_b1.reshape(1, H), out_w2, out_b2.reshape(1, H))
    hidden_new = hn2.reshape(B, N, H)

    flatT = pred2.reshape(B, N * H).T
    locT, sclT = _proj_tc(proj_loc_w.T, proj_scale_w.T, flatT,
                          proj_loc_b.reshape(N, 1), proj_scale_b.reshape(N, 1))
    loc = locT.T
    scale = sclT.T

    return ((loc, scale), hidden_new, edges)


# speedup vs baseline: 12.8679x; 1.0460x over previous
import functools

import numpy as np

import jax
import jax.numpy as jnp
from jax import lax
from jax.experimental import pallas as pl
from jax.experimental.pallas import tpu as pltpu
from jax.experimental.pallas import tpu_sc as plsc

B = 4
N = 1000
E = 64000
H = 64
IN = 8
L = 2
ET = 2
TAU = 0.5

BE = B * E
BEH = BE // 2

def _gumbel_const(shape):
    n = int(np.prod(shape))
    with np.errstate(over="ignore"):
        k0 = np.uint32(0)
        k1 = np.uint32(42)
        ks = [k0, k1, np.uint32(k0 ^ k1 ^ np.uint32(0x1BD11BDA))]
        rot = [(13, 15, 26, 6), (17, 29, 16, 24)]

        def rounds(a, b, rots):
            for r in rots:
                a = (a + b).astype(np.uint32)
                b = ((b << np.uint32(r)) | (b >> np.uint32(32 - r))).astype(np.uint32)
                b = a ^ b
            return a, b

        a = np.full(n, ks[0], np.uint32)
        b = (np.arange(n, dtype=np.uint32) + ks[1]).astype(np.uint32)
        a, b = rounds(a, b, rot[0])
        a = (a + ks[1]).astype(np.uint32); b = (b + ks[2] + np.uint32(1)).astype(np.uint32)
        a, b = rounds(a, b, rot[1])
        a = (a + ks[2]).astype(np.uint32); b = (b + ks[0] + np.uint32(2)).astype(np.uint32)
        a, b = rounds(a, b, rot[0])
        a = (a + ks[0]).astype(np.uint32); b = (b + ks[1] + np.uint32(3)).astype(np.uint32)
        a, b = rounds(a, b, rot[1])
        a = (a + ks[1]).astype(np.uint32); b = (b + ks[2] + np.uint32(4)).astype(np.uint32)
        a, b = rounds(a, b, rot[0])
        a = (a + ks[2]).astype(np.uint32); b = (b + ks[0] + np.uint32(5)).astype(np.uint32)
        bits = (a ^ b).reshape(shape)
    fb = ((bits >> np.uint32(9)) | np.uint32(0x3F800000)).view(np.float32)
    f = (fb - np.float32(1.0)).astype(np.float32)
    tiny = np.float32(np.finfo(np.float32).tiny)
    mm = np.float32(np.float32(1.0) - tiny)
    u = np.maximum(tiny, (f * mm + tiny).astype(np.float32))
    return (-np.log(-np.log(u))).astype(np.float32)


_GUMBEL = _gumbel_const((BE, ET))


def _edges_body(l0, l1, g0, g1, e0, e1, mb):
    a = (l0[...] + g0[...]) / TAU
    b = (l1[...] + g1[...]) / TAU
    m = jnp.maximum(a, b)
    ea = jnp.exp(a - m)
    eb = jnp.exp(b - m)
    s = ea + eb
    y0 = ea / s
    y1 = eb / s
    hard1 = b > a
    h1 = hard1.astype(jnp.float32)
    h0 = 1.0 - h1
    e0[...] = (h0 - y0) + y0
    e1[...] = (h1 - y1) + y1
    mb[...] = hard1.astype(jnp.int32)


def _edges_tc(l0, l1, g0, g1):
    return pl.pallas_call(
        _edges_body,
        out_shape=(
            jax.ShapeDtypeStruct((2000, 128), jnp.float32),
            jax.ShapeDtypeStruct((2000, 128), jnp.float32),
            jax.ShapeDtypeStruct((2000, 128), jnp.int32),
        ),
    )(l0, l1, g0, g1)


def _node0_body(h2, w1a, w1b, b1, x2, wr, br, wi, bi, wn, bn,
                hr, hs, ir, ii, inn):
    h = h2[...]
    hr[...] = jnp.dot(h, w1a[...], preferred_element_type=jnp.float32) + b1[...]
    hs[...] = jnp.dot(h, w1b[...], preferred_element_type=jnp.float32)
    x = x2[...]
    ir[...] = jnp.dot(x, wr[...], preferred_element_type=jnp.float32) + br[...]
    ii[...] = jnp.dot(x, wi[...], preferred_element_type=jnp.float32) + bi[...]
    inn[...] = jnp.dot(x, wn[...], preferred_element_type=jnp.float32) + bn[...]


def _node0_tc(h2, w1a, w1b, b1, x2, wr, br, wi, bi, wn, bn):
    f = jax.ShapeDtypeStruct
    return pl.pallas_call(
        _node0_body,
        out_shape=(
            f((B * N, H), jnp.float32), f((B * N, H), jnp.float32),
            f((B * N, H), jnp.float32), f((B * N, H), jnp.float32),
            f((B * N, H), jnp.float32),
        ),
    )(h2, w1a, w1b, b1, x2, wr, br, wi, bi, wn, bn)


_FC2_BLK = 4000


def _fc2_body(z, w2d, b2d, out):
    m = jnp.tanh(z[...])
    t = jnp.dot(m, w2d[...], preferred_element_type=jnp.float32) + b2d[...]
    out[...] = jnp.tanh(t)


def _fc2_tc(z, w2d, b2d):
    grid = z.shape[0] // _FC2_BLK
    return pl.pallas_call(
        _fc2_body,
        grid=(grid,),
        in_specs=[
            pl.BlockSpec((_FC2_BLK, 128), lambda i: (i, 0)),
            pl.BlockSpec((128, 128), lambda i: (0, 0)),
            pl.BlockSpec((1, 128), lambda i: (0, 0)),
        ],
        out_specs=pl.BlockSpec((_FC2_BLK, 128), lambda i: (i, 0)),
        out_shape=jax.ShapeDtypeStruct(z.shape, jnp.float32),
    )(z, w2d, b2d)


def _comb_body(pa, pb, dp, w1a, w1b, b1, agg0, hr, hs, dout):
    d = dp[0] + dp[1]
    dcol = d[:, 0:1]
    dcol = jnp.where(dcol == 0.0, 1.0, dcol)
    d4 = jnp.concatenate([dcol, dcol, dcol, dcol], axis=0)
    a0 = jnp.concatenate([pa[0] + pa[1], pb[0] + pb[1]], axis=0) / d4
    agg0[...] = a0
    hr[...] = jnp.dot(a0, w1a[...], preferred_element_type=jnp.float32) + b1[...]
    hs[...] = jnp.dot(a0, w1b[...], preferred_element_type=jnp.float32)
    dout[...] = dcol


def _comb_tc(pa, pb, dp, w1a, w1b, b1):
    f = jax.ShapeDtypeStruct
    return pl.pallas_call(
        _comb_body,
        out_shape=(
            f((B * N, H), jnp.float32), f((B * N, H), jnp.float32),
            f((B * N, H), jnp.float32), f((N, 1), jnp.float32),
        ),
    )(pa, pb, dp, w1a, w1b, b1)


def _gru_body(pa, pb, dcol, agg0, ir, ii, inn, h2, wr, wi, wh,
              ow1, ob1, ow2, ob2, hn_out, pred_out):
    dc = dcol[...]
    d4 = jnp.concatenate([dc, dc, dc, dc], axis=0)
    agg1 = jnp.concatenate([pa[0] + pa[1], pb[0] + pb[1]], axis=0) / d4
    am = jnp.concatenate([agg0[...], agg1], axis=1)
    r = jax.nn.sigmoid(ir[...] + jnp.dot(am, wr[...], preferred_element_type=jnp.float32))
    ig = jax.nn.sigmoid(ii[...] + jnp.dot(am, wi[...], preferred_element_type=jnp.float32))
    ng = jnp.tanh(inn[...] + r * jnp.dot(am, wh[...], preferred_element_type=jnp.float32))
    hn = (1.0 - ig) * ng + ig * h2[...]
    hn_out[...] = hn
    h1 = jax.nn.relu(jnp.dot(hn, ow1[...], preferred_element_type=jnp.float32) + ob1[...])
    pred_out[...] = jax.nn.relu(jnp.dot(h1, ow2[...], preferred_element_type=jnp.float32) + ob2[...])


def _gru_tc(pa, pb, dcol, agg0, ir, ii, inn, h2, wr, wi, wh, ow1, ob1, ow2, ob2):
    f = jax.ShapeDtypeStruct
    return pl.pallas_call(
        _gru_body,
        out_shape=(f((B * N, H), jnp.float32), f((B * N, H), jnp.float32)),
    )(pa, pb, dcol, agg0, ir, ii, inn, h2, wr, wi, wh, ow1, ob1, ow2, ob2)


_PROJ_KC = 1280


def _proj_body(wlt, wst, ft, bl, bs, loc, scl):
    i = pl.program_id(0)
    nsteps = pl.num_programs(0)
    fb = ft[...]
    pl_part = jnp.dot(wlt[...], fb, preferred_element_type=jnp.float32)
    ps_part = jnp.dot(wst[...], fb, preferred_element_type=jnp.float32)

    @pl.when(i == 0)
    def _():
        loc[...] = jnp.zeros_like(loc)
        scl[...] = jnp.zeros_like(scl)

    loc[...] += pl_part
    scl[...] += ps_part

    @pl.when(i == nsteps - 1)
    def _():
        loc[...] += bl[...]
        s = scl[...] + bs[...]
        scl[...] = jnp.log1p(jnp.exp(-jnp.abs(s))) + jax.nn.relu(s)


def _proj_tc(wlt, wst, ft, bl, bs):
    grid = (N * H) // _PROJ_KC
    f = jax.ShapeDtypeStruct
    return pl.pallas_call(
        _proj_body,
        grid=(grid,),
        in_specs=[
            pl.BlockSpec((N, _PROJ_KC), lambda i: (0, i)),
            pl.BlockSpec((N, _PROJ_KC), lambda i: (0, i)),
            pl.BlockSpec((_PROJ_KC, B), lambda i: (i, 0)),
            pl.BlockSpec((N, 1), lambda i: (0, 0)),
            pl.BlockSpec((N, 1), lambda i: (0, 0)),
        ],
        out_specs=(
            pl.BlockSpec((N, B), lambda i: (0, 0)),
            pl.BlockSpec((N, B), lambda i: (0, 0)),
        ),
        out_shape=(f((N, B), jnp.float32), f((N, B), jnp.float32)),
    )(wlt, wst, ft, bl, bs)


_NC = 2
_NS = 16
_NW = _NC * _NS
_EPT = E // _NW
_CH = 400
_NCH = _EPT // _CH
_SUB = 80
_NSUB = _CH // _SUB

_sc_mesh = plsc.VectorSubcoreMesh(core_axis_name="c", subcore_axis_name="s")


def _make_sc_gather(do_deg, b_lo):
    out_type = [
        jax.ShapeDtypeStruct((2 * E, H), jnp.float32),
    ]
    scratch = [
        pltpu.VMEM((_EPT,), jnp.int32),
        pltpu.VMEM((_EPT,), jnp.int32),
        pltpu.VMEM((2, _NSUB, _SUB), jnp.int32),
        pltpu.VMEM((2, _NSUB, _SUB), jnp.int32),
        pltpu.VMEM((2, _CH, H), jnp.float32),
        pltpu.VMEM((2, _CH, H), jnp.float32),
        pltpu.SemaphoreType.DMA,
        pltpu.SemaphoreType.DMA,
    ]
    if do_deg:
        out_type.append(jax.ShapeDtypeStruct((2, 1024, 16), jnp.float32))
        scratch.append(pltpu.VMEM((_SUB, 16), jnp.float32))
        scratch.append(pltpu.VMEM_SHARED((1024, 16), jnp.float32))

    def body(*refs):
        if do_deg:
            (tabr, tabs, recv, send, ones_h, zer_h, z, degp,
             rawr, raws, idx2r, idx2s, bufr, bufs, sem0, sem1, onesv, dacc) = refs
        else:
            (tabr, tabs, recv, send, z,
             rawr, raws, idx2r, idx2s, bufr, bufs, sem0, sem1) = refs
        sems = (sem0, sem1)
        c = lax.axis_index("c")
        s = lax.axis_index("s")
        w = c * _NS + s
        e_base = w * _EPT
        pltpu.sync_copy(recv.at[pl.ds(e_base, _EPT)], rawr)
        pltpu.sync_copy(send.at[pl.ds(e_base, _EPT)], raws)
        if do_deg:
            pltpu.sync_copy(ones_h, onesv)
            pltpu.sync_copy(zer_h, dacc.at[pl.ds(s * 64, 64)])
            plsc.subcore_barrier()

        iters = [(bi, ch) for bi in range(2) for ch in range(_NCH)]

        def stage(i):
            bi, ch = iters[i]
            sl = i % 2
            for j in range(_NSUB):
                for k in range(_SUB // 16):
                    srcs = pl.ds(ch * _CH + j * _SUB + k * 16, 16)
                    dst = pl.ds(k * 16, 16)
                    idx2r[sl, j, dst] = rawr[srcs] + (b_lo + bi) * N
                    idx2s[sl, j, dst] = raws[srcs] + (b_lo + bi) * N
            cps = []
            for j in range(_NSUB):
                d = pl.ds(j * _SUB, _SUB)
                cps.append(pltpu.async_copy(tabr.at[idx2r.at[sl, j]], bufr.at[sl, d], sems[sl]))
                cps.append(pltpu.async_copy(tabs.at[idx2s.at[sl, j]], bufs.at[sl, d], sems[sl]))
            return cps

        pend = stage(0)
        for i in range(len(iters)):
            bi, ch = iters[i]
            sl = i % 2
            nxt_pend = stage(i + 1) if i + 1 < len(iters) else []
            for cp in pend:
                cp.wait()

            @plsc.parallel_loop(0, _CH, step=1, unroll=8)
            def _add(r):
                for k in range(H // 16):
                    slc = pl.ds(k * 16, 16)
                    bufr[sl, r, slc] = bufr[sl, r, slc] + bufs[sl, r, slc]

            row0 = bi * E + e_base + ch * _CH
            pltpu.sync_copy(bufr.at[sl], z.at[pl.ds(row0, _CH)])
            if do_deg and b_lo + bi == 0:
                for j in range(_NSUB):
                    pltpu.sync_copy(onesv, dacc.at[idx2r.at[sl, j]], add=True)
            pend = nxt_pend
        if do_deg:
            plsc.subcore_barrier()
            pltpu.sync_copy(dacc.at[pl.ds(s * 64, 64)], degp.at[c, pl.ds(s * 64, 64)])

    kw = dict(out_type=tuple(out_type), mesh=_sc_mesh,
              compiler_params=pltpu.CompilerParams(use_tc_tiling_on_sc=False),
              scratch_types=scratch)
    return functools.partial(pl.kernel, **kw)(body)


_sc_gather_a = _make_sc_gather(True, 0)
_sc_gather_b = _make_sc_gather(False, 2)
_sc_gather_a1 = _make_sc_gather(False, 0)


def _make_sc_scatter(b_lo):
    def body(m2, recv, mb, zer_h, pout, rawr, rawm, idx2, bufm, macc,
             sem0, sem1, sem2, sem3):
        sems = (sem0, sem1)
        asems = (sem2, sem3)
        c = lax.axis_index("c")
        s = lax.axis_index("s")
        w = c * _NS + s
        e_base = w * _EPT
        pltpu.sync_copy(recv.at[pl.ds(e_base, _EPT)], rawr)
        pltpu.sync_copy(zer_h, macc.at[pl.ds(s * 128, 128)])
        plsc.subcore_barrier()

        iters = [(bi, ch) for bi in range(2) for ch in range(_NCH)]

        def stage(i):
            bi, ch = iters[i]
            sl = i % 2
            rloc = bi * E + e_base + ch * _CH
            rglob = (b_lo + bi) * E + e_base + ch * _CH
            pltpu.sync_copy(mb.at[pl.ds(rglob, _CH)], rawm.at[pl.ds(sl * _CH, _CH)])
            cp = pltpu.async_copy(m2.at[pl.ds(rloc, _CH)], bufm.at[sl], sems[sl])
            for j in range(_NSUB):
                for k in range(_SUB // 16):
                    msl = pl.ds(sl * _CH + j * _SUB + k * 16, 16)
                    esl = pl.ds(ch * _CH + j * _SUB + k * 16, 16)
                    idx2[sl, j, pl.ds(k * 16, 16)] = (
                        rawm[msl] * (rawr[esl] - 1000) + (1000 + bi * 1024))
            return cp

        addp = {0: [], 1: []}
        pend = stage(0)
        for i in range(len(iters)):
            sl = i % 2
            if i + 1 < len(iters):
                sln = (i + 1) % 2
                for cp in addp[sln]:
                    cp.wait()
                addp[sln] = []
                nxt = stage(i + 1)
            else:
                nxt = None
            pend.wait()
            addp[sl] = [
                pltpu.async_copy(bufm.at[sl, pl.ds(j * _SUB, _SUB)],
                                 macc.at[idx2.at[sl, j]], asems[sl], add=True)
                for j in range(_NSUB)]
            pend = nxt
        for sl in (0, 1):
            for cp in addp[sl]:
                cp.wait()
        plsc.subcore_barrier()
        pltpu.sync_copy(macc.at[pl.ds(s * 128, 128)], pout.at[c, pl.ds(s * 128, 128)])

    return functools.partial(
        pl.kernel,
        out_type=jax.ShapeDtypeStruct((2, 2048, H), jnp.float32),
        mesh=_sc_mesh,
        compiler_params=pltpu.CompilerParams(use_tc_tiling_on_sc=False),
        scratch_types=[
            pltpu.VMEM((_EPT,), jnp.int32),
            pltpu.VMEM((2 * _CH,), jnp.int32),
            pltpu.VMEM((2, _NSUB, _SUB), jnp.int32),
            pltpu.VMEM((2, _CH, H), jnp.float32),
            pltpu.VMEM_SHARED((2048, H), jnp.float32),
            pltpu.SemaphoreType.DMA,
            pltpu.SemaphoreType.DMA,
            pltpu.SemaphoreType.DMA,
            pltpu.SemaphoreType.DMA,
        ],
    )(body)


_sc_scatter_a = _make_sc_scatter(0)
_sc_scatter_b = _make_sc_scatter(2)


def kernel(inputs, hidden, edge_logits, send_edges, recv_edges,
           msg_fc1_w, msg_fc1_b, msg_fc2_w, msg_fc2_b,
           hidden_r_w, hidden_i_w, hidden_h_w,
           input_r_w, input_r_b, input_i_w, input_i_b, input_n_w, input_n_b,
           out_w1, out_b1, out_w2, out_b2,
           proj_loc_w, proj_loc_b, proj_scale_w, proj_scale_b):
    f32 = jnp.float32

    g = _GUMBEL
    l0 = edge_logits[:, :, 0].reshape(2000, 128)
    l1 = edge_logits[:, :, 1].reshape(2000, 128)
    g0 = jnp.asarray(g[:, 0].reshape(2000, 128))
    g1 = jnp.asarray(g[:, 1].reshape(2000, 128))
    e0, e1, mbin = _edges_tc(l0, l1, g0, g1)
    edges = jnp.stack([e0.reshape(B, E), e1.reshape(B, E)], axis=-1)
    mb1d = mbin.reshape(BE)

    h2 = hidden.reshape(B * N, H)
    x2 = inputs.reshape(B * N, IN)
    w1 = msg_fc1_w[0, 1]
    hr0, hs0, ir, ii, inn = _node0_tc(
        h2, w1[:H], w1[H:], msg_fc1_b[0, 1].reshape(1, H), x2,
        input_r_w, input_r_b.reshape(1, H),
        input_i_w, input_i_b.reshape(1, H),
        input_n_w, input_n_b.reshape(1, H))

    ones16 = jnp.ones((_SUB, 16), f32)
    zer16 = jnp.zeros((64, 16), f32)
    zer64 = jnp.zeros((128, H), f32)

    def w2diag(w2, b2):
        wd = jnp.zeros((128, 128), f32)
        wd = wd.at[:H, :H].set(w2).at[H:, H:].set(w2)
        bd = jnp.concatenate([b2, b2]).reshape(1, 128)
        return wd, bd

    w2d0, b2d0 = w2diag(msg_fc2_w[0, 1], msg_fc2_b[0, 1])
    w2d1, b2d1 = w2diag(msg_fc2_w[1, 1], msg_fc2_b[1, 1])

    zA, degp4 = _sc_gather_a(hr0, hs0, recv_edges, send_edges, ones16, zer16)
    zB, = _sc_gather_b(hr0, hs0, recv_edges, send_edges)
    m2A = _fc2_tc(zA.reshape(E, 128), w2d0, b2d0)
    m2B = _fc2_tc(zB.reshape(E, 128), w2d0, b2d0)
    pA = _sc_scatter_a(m2A.reshape(2 * E, H), recv_edges, mb1d, zer64)
    pB = _sc_scatter_b(m2B.reshape(2 * E, H), recv_edges, mb1d, zer64)
    pa0 = pA.reshape(2, 2, 1024, H)[:, :, :N, :].reshape(2, 2 * N, H)
    pb0 = pB.reshape(2, 2, 1024, H)[:, :, :N, :].reshape(2, 2 * N, H)
    degp = degp4[:, :N, :]

    w1_1 = msg_fc1_w[1, 1]
    agg0, hr1, hs1, dcol = _comb_tc(
        pa0, pb0, degp, w1_1[:H], w1_1[H:], msg_fc1_b[1, 1].reshape(1, H))

    zA1, = _sc_gather_a1(hr1, hs1, recv_edges, send_edges)
    zB1, = _sc_gather_b(hr1, hs1, recv_edges, send_edges)
    m2A1 = _fc2_tc(zA1.reshape(E, 128), w2d1, b2d1)
    m2B1 = _fc2_tc(zB1.reshape(E, 128), w2d1, b2d1)
    pA1 = _sc_scatter_a(m2A1.reshape(2 * E, H), recv_edges, mb1d, zer64)
    pB1 = _sc_scatter_b(m2B1.reshape(2 * E, H), recv_edges, mb1d, zer64)
    pa1 = pA1.reshape(2, 2, 1024, H)[:, :, :N, :].reshape(2, 2 * N, H)
    pb1 = pB1.reshape(2, 2, 1024, H)[:, :, :N, :].reshape(2, 2 * N, H)

    hn2, pred2 = _gru_tc(pa1, pb1, dcol, agg0, ir, ii, inn, h2,
                         hidden_r_w, hidden_i_w, hidden_h_w,
                         out_w1, out_b1.reshape(1, H), out_w2, out_b2.reshape(1, H))
    hidden_new = hn2.reshape(B, N, H)

    flatT = pred2.reshape(B, N * H).T
    locT, sclT = _proj_tc(proj_loc_w.T, proj_scale_w.T, flatT,
                          proj_loc_b.reshape(N, 1), proj_scale_b.reshape(N, 1))
    loc = locT.T
    scale = sclT.T

    return ((loc, scale), hidden_new, edges)


# speedup vs baseline: 12.8917x; 1.0018x over previous
import functools

import numpy as np

import jax
import jax.numpy as jnp
from jax import lax
from jax.experimental import pallas as pl
from jax.experimental.pallas import tpu as pltpu
from jax.experimental.pallas import tpu_sc as plsc

B = 4
N = 1000
E = 64000
H = 64
IN = 8
L = 2
ET = 2
TAU = 0.5

BE = B * E

def _gumbel_const(shape):
    n = int(np.prod(shape))
    with np.errstate(over="ignore"):
        k0 = np.uint32(0)
        k1 = np.uint32(42)
        ks = [k0, k1, np.uint32(k0 ^ k1 ^ np.uint32(0x1BD11BDA))]
        rot = [(13, 15, 26, 6), (17, 29, 16, 24)]

        def rounds(a, b, rots):
            for r in rots:
                a = (a + b).astype(np.uint32)
                b = ((b << np.uint32(r)) | (b >> np.uint32(32 - r))).astype(np.uint32)
                b = a ^ b
            return a, b

        a = np.full(n, ks[0], np.uint32)
        b = (np.arange(n, dtype=np.uint32) + ks[1]).astype(np.uint32)
        a, b = rounds(a, b, rot[0])
        a = (a + ks[1]).astype(np.uint32); b = (b + ks[2] + np.uint32(1)).astype(np.uint32)
        a, b = rounds(a, b, rot[1])
        a = (a + ks[2]).astype(np.uint32); b = (b + ks[0] + np.uint32(2)).astype(np.uint32)
        a, b = rounds(a, b, rot[0])
        a = (a + ks[0]).astype(np.uint32); b = (b + ks[1] + np.uint32(3)).astype(np.uint32)
        a, b = rounds(a, b, rot[1])
        a = (a + ks[1]).astype(np.uint32); b = (b + ks[2] + np.uint32(4)).astype(np.uint32)
        a, b = rounds(a, b, rot[0])
        a = (a + ks[2]).astype(np.uint32); b = (b + ks[0] + np.uint32(5)).astype(np.uint32)
        bits = (a ^ b).reshape(shape)
    fb = ((bits >> np.uint32(9)) | np.uint32(0x3F800000)).view(np.float32)
    f = (fb - np.float32(1.0)).astype(np.float32)
    tiny = np.float32(np.finfo(np.float32).tiny)
    mm = np.float32(np.float32(1.0) - tiny)
    u = np.maximum(tiny, (f * mm + tiny).astype(np.float32))
    return (-np.log(-np.log(u))).astype(np.float32)


_GUMBEL = _gumbel_const((BE, ET))


def _edges_body(l0, l1, g0, g1, e0, e1, mb):
    a = (l0[...] + g0[...]) / TAU
    b = (l1[...] + g1[...]) / TAU
    m = jnp.maximum(a, b)
    ea = jnp.exp(a - m)
    eb = jnp.exp(b - m)
    s = ea + eb
    y0 = ea / s
    y1 = eb / s
    hard1 = b > a
    h1 = hard1.astype(jnp.float32)
    h0 = 1.0 - h1
    e0[...] = (h0 - y0) + y0
    e1[...] = (h1 - y1) + y1
    mb[...] = hard1.astype(jnp.int32)


def _edges_tc(l0, l1, g0, g1):
    return pl.pallas_call(
        _edges_body,
        out_shape=(
            jax.ShapeDtypeStruct((2000, 128), jnp.float32),
            jax.ShapeDtypeStruct((2000, 128), jnp.float32),
            jax.ShapeDtypeStruct((2000, 128), jnp.int32),
        ),
    )(l0, l1, g0, g1)


def _node0_body(h2, w1a, w1b, b1, x2, wr, br, wi, bi, wn, bn,
                hr, hs, ir, ii, inn):
    h = h2[...]
    hr[...] = jnp.dot(h, w1a[...], preferred_element_type=jnp.float32) + b1[...]
    hs[...] = jnp.dot(h, w1b[...], preferred_element_type=jnp.float32)
    x = x2[...]
    ir[...] = jnp.dot(x, wr[...], preferred_element_type=jnp.float32) + br[...]
    ii[...] = jnp.dot(x, wi[...], preferred_element_type=jnp.float32) + bi[...]
    inn[...] = jnp.dot(x, wn[...], preferred_element_type=jnp.float32) + bn[...]


def _node0_tc(h2, w1a, w1b, b1, x2, wr, br, wi, bi, wn, bn):
    f = jax.ShapeDtypeStruct
    return pl.pallas_call(
        _node0_body,
        out_shape=(
            f((B * N, H), jnp.float32), f((B * N, H), jnp.float32),
            f((B * N, H), jnp.float32), f((B * N, H), jnp.float32),
            f((B * N, H), jnp.float32),
        ),
    )(h2, w1a, w1b, b1, x2, wr, br, wi, bi, wn, bn)


_FC2_BLK = 4000


def _fc2_body(z, w2d, b2d, out):
    m = jnp.tanh(z[...])
    t = jnp.dot(m, w2d[...], preferred_element_type=jnp.float32) + b2d[...]
    out[...] = jnp.tanh(t)


def _fc2_tc(z, w2d, b2d):
    grid = z.shape[0] // _FC2_BLK
    return pl.pallas_call(
        _fc2_body,
        grid=(grid,),
        in_specs=[
            pl.BlockSpec((_FC2_BLK, 128), lambda i: (i, 0)),
            pl.BlockSpec((128, 128), lambda i: (0, 0)),
            pl.BlockSpec((1, 128), lambda i: (0, 0)),
        ],
        out_specs=pl.BlockSpec((_FC2_BLK, 128), lambda i: (i, 0)),
        out_shape=jax.ShapeDtypeStruct(z.shape, jnp.float32),
    )(z, w2d, b2d)


def _comb_body(pa, pb, dp, w1a, w1b, b1, agg0, hr, hs, dout):
    d = dp[0] + dp[1]
    dcol = d[:, 0:1]
    dcol = jnp.where(dcol == 0.0, 1.0, dcol)
    d4 = jnp.concatenate([dcol, dcol, dcol, dcol], axis=0)
    a0 = jnp.concatenate([pa[0] + pa[1], pb[0] + pb[1]], axis=0) / d4
    agg0[...] = a0
    hr[...] = jnp.dot(a0, w1a[...], preferred_element_type=jnp.float32) + b1[...]
    hs[...] = jnp.dot(a0, w1b[...], preferred_element_type=jnp.float32)
    dout[...] = dcol


def _comb_tc(pa, pb, dp, w1a, w1b, b1):
    f = jax.ShapeDtypeStruct
    return pl.pallas_call(
        _comb_body,
        out_shape=(
            f((B * N, H), jnp.float32), f((B * N, H), jnp.float32),
            f((B * N, H), jnp.float32), f((N, 1), jnp.float32),
        ),
    )(pa, pb, dp, w1a, w1b, b1)


def _gru_body(pa, pb, dcol, agg0, ir, ii, inn, h2, wr, wi, wh,
              ow1, ob1, ow2, ob2, hn_out, pred_out):
    dc = dcol[...]
    d4 = jnp.concatenate([dc, dc, dc, dc], axis=0)
    agg1 = jnp.concatenate([pa[0] + pa[1], pb[0] + pb[1]], axis=0) / d4
    am = jnp.concatenate([agg0[...], agg1], axis=1)
    r = jax.nn.sigmoid(ir[...] + jnp.dot(am, wr[...], preferred_element_type=jnp.float32))
    ig = jax.nn.sigmoid(ii[...] + jnp.dot(am, wi[...], preferred_element_type=jnp.float32))
    ng = jnp.tanh(inn[...] + r * jnp.dot(am, wh[...], preferred_element_type=jnp.float32))
    hn = (1.0 - ig) * ng + ig * h2[...]
    hn_out[...] = hn
    h1 = jax.nn.relu(jnp.dot(hn, ow1[...], preferred_element_type=jnp.float32) + ob1[...])
    pred_out[...] = jax.nn.relu(jnp.dot(h1, ow2[...], preferred_element_type=jnp.float32) + ob2[...])


def _gru_tc(pa, pb, dcol, agg0, ir, ii, inn, h2, wr, wi, wh, ow1, ob1, ow2, ob2):
    f = jax.ShapeDtypeStruct
    return pl.pallas_call(
        _gru_body,
        out_shape=(f((B * N, H), jnp.float32), f((B * N, H), jnp.float32)),
    )(pa, pb, dcol, agg0, ir, ii, inn, h2, wr, wi, wh, ow1, ob1, ow2, ob2)


_PROJ_KC = 1280


def _proj_body(wlt, wst, ft, bl, bs, loc, scl):
    i = pl.program_id(0)
    nsteps = pl.num_programs(0)
    fb = ft[...]
    pl_part = jnp.dot(wlt[...], fb, preferred_element_type=jnp.float32)
    ps_part = jnp.dot(wst[...], fb, preferred_element_type=jnp.float32)

    @pl.when(i == 0)
    def _():
        loc[...] = jnp.zeros_like(loc)
        scl[...] = jnp.zeros_like(scl)

    loc[...] += pl_part
    scl[...] += ps_part

    @pl.when(i == nsteps - 1)
    def _():
        loc[...] += bl[...]
        s = scl[...] + bs[...]
        scl[...] = jnp.log1p(jnp.exp(-jnp.abs(s))) + jax.nn.relu(s)


def _proj_tc(wlt, wst, ft, bl, bs):
    grid = (N * H) // _PROJ_KC
    f = jax.ShapeDtypeStruct
    return pl.pallas_call(
        _proj_body,
        grid=(grid,),
        in_specs=[
            pl.BlockSpec((N, _PROJ_KC), lambda i: (0, i)),
            pl.BlockSpec((N, _PROJ_KC), lambda i: (0, i)),
            pl.BlockSpec((_PROJ_KC, B), lambda i: (i, 0)),
            pl.BlockSpec((N, 1), lambda i: (0, 0)),
            pl.BlockSpec((N, 1), lambda i: (0, 0)),
        ],
        out_specs=(
            pl.BlockSpec((N, B), lambda i: (0, 0)),
            pl.BlockSpec((N, B), lambda i: (0, 0)),
        ),
        out_shape=(f((N, B), jnp.float32), f((N, B), jnp.float32)),
    )(wlt, wst, ft, bl, bs)


_NC = 2
_NS = 16
_NW = _NC * _NS
_EPT = E // _NW
_CH = 400
_NCH = _EPT // _CH
_SUB = 80
_NSUB = _CH // _SUB

_sc_mesh = plsc.VectorSubcoreMesh(core_axis_name="c", subcore_axis_name="s")


def _make_sc_gather(do_deg, b_lo):
    out_type = [
        jax.ShapeDtypeStruct((2 * E, H), jnp.float32),
    ]
    scratch = [
        pltpu.VMEM((_EPT,), jnp.int32),
        pltpu.VMEM((_EPT,), jnp.int32),
        pltpu.VMEM((2, _NSUB, _SUB), jnp.int32),
        pltpu.VMEM((2, _NSUB, _SUB), jnp.int32),
        pltpu.VMEM((2, _CH, H), jnp.float32),
        pltpu.VMEM((2, _CH, H), jnp.float32),
        pltpu.SemaphoreType.DMA,
        pltpu.SemaphoreType.DMA,
    ]
    if do_deg:
        out_type.append(jax.ShapeDtypeStruct((2, 1024, 16), jnp.float32))
        scratch.append(pltpu.VMEM((_SUB, 16), jnp.float32))
        scratch.append(pltpu.VMEM_SHARED((1024, 16), jnp.float32))

    def body(*refs):
        if do_deg:
            (tabr, tabs, recv, send, ones_h, zer_h, z, degp,
             rawr, raws, idx2r, idx2s, bufr, bufs, sem0, sem1, onesv, dacc) = refs
        else:
            (tabr, tabs, recv, send, z,
             rawr, raws, idx2r, idx2s, bufr, bufs, sem0, sem1) = refs
        sems = (sem0, sem1)
        c = lax.axis_index("c")
        s = lax.axis_index("s")
        w = c * _NS + s
        e_base = w * _EPT
        pltpu.sync_copy(recv.at[pl.ds(e_base, _EPT)], rawr)
        pltpu.sync_copy(send.at[pl.ds(e_base, _EPT)], raws)
        if do_deg:
            pltpu.sync_copy(ones_h, onesv)
            pltpu.sync_copy(zer_h, dacc.at[pl.ds(s * 64, 64)])
            plsc.subcore_barrier()

        iters = [(bi, ch) for bi in range(2) for ch in range(_NCH)]

        def stage(i):
            bi, ch = iters[i]
            sl = i % 2
            for j in range(_NSUB):
                for k in range(_SUB // 16):
                    srcs = pl.ds(ch * _CH + j * _SUB + k * 16, 16)
                    dst = pl.ds(k * 16, 16)
                    idx2r[sl, j, dst] = rawr[srcs] + (b_lo + bi) * N
                    idx2s[sl, j, dst] = raws[srcs] + (b_lo + bi) * N
            cps = []
            for j in range(_NSUB):
                d = pl.ds(j * _SUB, _SUB)
                cps.append(pltpu.async_copy(tabr.at[idx2r.at[sl, j]], bufr.at[sl, d], sems[sl]))
                cps.append(pltpu.async_copy(tabs.at[idx2s.at[sl, j]], bufs.at[sl, d], sems[sl]))
            return cps

        pend = stage(0)
        for i in range(len(iters)):
            bi, ch = iters[i]
            sl = i % 2
            nxt_pend = stage(i + 1) if i + 1 < len(iters) else []
            for cp in pend:
                cp.wait()

            @plsc.parallel_loop(0, _CH, step=1, unroll=8)
            def _add(r):
                for k in range(H // 16):
                    slc = pl.ds(k * 16, 16)
                    bufr[sl, r, slc] = bufr[sl, r, slc] + bufs[sl, r, slc]

            row0 = bi * E + e_base + ch * _CH
            pltpu.sync_copy(bufr.at[sl], z.at[pl.ds(row0, _CH)])
            if do_deg and b_lo + bi == 0:
                for j in range(_NSUB):
                    pltpu.sync_copy(onesv, dacc.at[idx2r.at[sl, j]], add=True)
            pend = nxt_pend
        if do_deg:
            plsc.subcore_barrier()
            pltpu.sync_copy(dacc.at[pl.ds(s * 64, 64)], degp.at[c, pl.ds(s * 64, 64)])

    kw = dict(out_type=tuple(out_type), mesh=_sc_mesh,
              compiler_params=pltpu.CompilerParams(use_tc_tiling_on_sc=False),
              scratch_types=scratch)
    return functools.partial(pl.kernel, **kw)(body)


_sc_gather_a = _make_sc_gather(True, 0)
_sc_gather_b = _make_sc_gather(False, 2)
_sc_gather_a1 = _make_sc_gather(False, 0)


def _make_sc_scatter(b_lo):
    def body(m2, recv, mb, zer_h, pout, rawr, rawm, idx2, bufm, macc,
             sem0, sem1, sem2, sem3):
        sems = (sem0, sem1)
        asems = (sem2, sem3)
        c = lax.axis_index("c")
        s = lax.axis_index("s")
        w = c * _NS + s
        e_base = w * _EPT
        pltpu.sync_copy(recv.at[pl.ds(e_base, _EPT)], rawr)
        pltpu.sync_copy(zer_h, macc.at[pl.ds(s * 128, 128)])
        plsc.subcore_barrier()

        iters = [(bi, ch) for bi in range(2) for ch in range(_NCH)]

        def stage(i):
            bi, ch = iters[i]
            sl = i % 2
            rloc = bi * E + e_base + ch * _CH
            rglob = (b_lo + bi) * E + e_base + ch * _CH
            pltpu.sync_copy(mb.at[pl.ds(rglob, _CH)], rawm.at[pl.ds(sl * _CH, _CH)])
            cp = pltpu.async_copy(m2.at[pl.ds(rloc, _CH)], bufm.at[sl], sems[sl])
            for j in range(_NSUB):
                for k in range(_SUB // 16):
                    msl = pl.ds(sl * _CH + j * _SUB + k * 16, 16)
                    esl = pl.ds(ch * _CH + j * _SUB + k * 16, 16)
                    idx2[sl, j, pl.ds(k * 16, 16)] = (
                        rawm[msl] * (rawr[esl] - 1000) + (1000 + bi * 1024))
            return cp

        addp = {0: [], 1: []}
        pend = stage(0)
        for i in range(len(iters)):
            sl = i % 2
            if i + 1 < len(iters):
                sln = (i + 1) % 2
                for cp in addp[sln]:
                    cp.wait()
                addp[sln] = []
                nxt = stage(i + 1)
            else:
                nxt = None
            pend.wait()
            addp[sl] = [
                pltpu.async_copy(bufm.at[sl, pl.ds(j * _SUB, _SUB)],
                                 macc.at[idx2.at[sl, j]], asems[sl], add=True)
                for j in range(_NSUB)]
            pend = nxt
        for sl in (0, 1):
            for cp in addp[sl]:
                cp.wait()
        plsc.subcore_barrier()
        pltpu.sync_copy(macc.at[pl.ds(s * 128, 128)], pout.at[c, pl.ds(s * 128, 128)])

    return functools.partial(
        pl.kernel,
        out_type=jax.ShapeDtypeStruct((2, 2048, H), jnp.float32),
        mesh=_sc_mesh,
        compiler_params=pltpu.CompilerParams(use_tc_tiling_on_sc=False),
        scratch_types=[
            pltpu.VMEM((_EPT,), jnp.int32),
            pltpu.VMEM((2 * _CH,), jnp.int32),
            pltpu.VMEM((2, _NSUB, _SUB), jnp.int32),
            pltpu.VMEM((2, _CH, H), jnp.float32),
            pltpu.VMEM_SHARED((2048, H), jnp.float32),
            pltpu.SemaphoreType.DMA,
            pltpu.SemaphoreType.DMA,
            pltpu.SemaphoreType.DMA,
            pltpu.SemaphoreType.DMA,
        ],
    )(body)


_sc_scatter_a = _make_sc_scatter(0)
_sc_scatter_b = _make_sc_scatter(2)


def kernel(inputs, hidden, edge_logits, send_edges, recv_edges,
           msg_fc1_w, msg_fc1_b, msg_fc2_w, msg_fc2_b,
           hidden_r_w, hidden_i_w, hidden_h_w,
           input_r_w, input_r_b, input_i_w, input_i_b, input_n_w, input_n_b,
           out_w1, out_b1, out_w2, out_b2,
           proj_loc_w, proj_loc_b, proj_scale_w, proj_scale_b):
    f32 = jnp.float32

    g = _GUMBEL
    l0 = edge_logits[:, :, 0].reshape(2000, 128)
    l1 = edge_logits[:, :, 1].reshape(2000, 128)
    g0 = jnp.asarray(g[:, 0].reshape(2000, 128))
    g1 = jnp.asarray(g[:, 1].reshape(2000, 128))
    e0, e1, mbin = _edges_tc(l0, l1, g0, g1)
    edges = jnp.stack([e0.reshape(B, E), e1.reshape(B, E)], axis=-1)
    mb1d = mbin.reshape(BE)

    h2 = hidden.reshape(B * N, H)
    x2 = inputs.reshape(B * N, IN)
    w1 = msg_fc1_w[0, 1]
    hr0, hs0, ir, ii, inn = _node0_tc(
        h2, w1[:H], w1[H:], msg_fc1_b[0, 1].reshape(1, H), x2,
        input_r_w, input_r_b.reshape(1, H),
        input_i_w, input_i_b.reshape(1, H),
        input_n_w, input_n_b.reshape(1, H))

    ones16 = jnp.ones((_SUB, 16), f32)
    zer16 = jnp.zeros((64, 16), f32)
    zer64 = jnp.zeros((128, H), f32)

    def w2diag(w2, b2):
        wd = jnp.zeros((128, 128), f32)
        wd = wd.at[:H, :H].set(w2).at[H:, H:].set(w2)
        bd = jnp.concatenate([b2, b2]).reshape(1, 128)
        return wd, bd

    w2d0, b2d0 = w2diag(msg_fc2_w[0, 1], msg_fc2_b[0, 1])
    w2d1, b2d1 = w2diag(msg_fc2_w[1, 1], msg_fc2_b[1, 1])

    zA, degp4 = _sc_gather_a(hr0, hs0, recv_edges, send_edges, ones16, zer16)
    zB, = _sc_gather_b(hr0, hs0, recv_edges, send_edges)
    m2A = _fc2_tc(zA.reshape(E, 128), w2d0, b2d0)
    m2B = _fc2_tc(zB.reshape(E, 128), w2d0, b2d0)
    pA = _sc_scatter_a(m2A.reshape(2 * E, H), recv_edges, mb1d, zer64)
    pB = _sc_scatter_b(m2B.reshape(2 * E, H), recv_edges, mb1d, zer64)
    pa0 = pA.reshape(2, 2, 1024, H)[:, :, :N, :].reshape(2, 2 * N, H)
    pb0 = pB.reshape(2, 2, 1024, H)[:, :, :N, :].reshape(2, 2 * N, H)
    degp = degp4[:, :N, :]

    w1_1 = msg_fc1_w[1, 1]
    agg0, hr1, hs1, dcol = _comb_tc(
        pa0, pb0, degp, w1_1[:H], w1_1[H:], msg_fc1_b[1, 1].reshape(1, H))

    zA1, = _sc_gather_a1(hr1, hs1, recv_edges, send_edges)
    zB1, = _sc_gather_b(hr1, hs1, recv_edges, send_edges)
    m2A1 = _fc2_tc(zA1.reshape(E, 128), w2d1, b2d1)
    m2B1 = _fc2_tc(zB1.reshape(E, 128), w2d1, b2d1)
    pA1 = _sc_scatter_a(m2A1.reshape(2 * E, H), recv_edges, mb1d, zer64)
    pB1 = _sc_scatter_b(m2B1.reshape(2 * E, H), recv_edges, mb1d, zer64)
    pa1 = pA1.reshape(2, 2, 1024, H)[:, :, :N, :].reshape(2, 2 * N, H)
    pb1 = pB1.reshape(2, 2, 1024, H)[:, :, :N, :].reshape(2, 2 * N, H)

    hn2, pred2 = _gru_tc(pa1, pb1, dcol, agg0, ir, ii, inn, h2,
                         hidden_r_w, hidden_i_w, hidden_h_w,
                         out_w1, out_b1.reshape(1, H), out_w2, out_b2.reshape(1, H))
    hidden_new = hn2.reshape(B, N, H)

    flatT = pred2.reshape(B, N * H).T
    locT, sclT = _proj_tc(proj_loc_w.T, proj_scale_w.T, flatT,
                          proj_loc_b.reshape(N, 1), proj_scale_b.reshape(N, 1))
    loc = locT.T
    scale = sclT.T

    return ((loc, scale), hidden_new, edges)
